# Initial kernel scaffold; baseline (speedup 1.0000x reference)
import jax
import jax.numpy as jnp
from jax.experimental import pallas as pl


def kernel(x, edge_index, weight, bias):
    raise NotImplementedError("write your pallas kernel here")


# speedup vs baseline: 241.2812x; 241.2812x over previous
import functools

import jax
import jax.numpy as jnp
from jax import lax
from jax.experimental import pallas as pl
from jax.experimental.pallas import tpu as pltpu
from jax.experimental.pallas import tpu_sc as plsc

N_NODES = 50000
N_PAD = 50176
TRASH = N_NODES
STRIPE = N_PAD // 16
E_EDGES = 800000
E_PAD = 819200
NC, NS = 2, 16
NW = NC * NS
EDGES_W = E_PAD // NW
CHUNK = 128
NCHUNK = EDGES_W // CHUNK
F_IN = 64
F_OUT = 16

_mesh = plsc.VectorSubcoreMesh(core_axis_name="c", subcore_axis_name="s")


def _wid():
    return lax.axis_index("s") * NC + lax.axis_index("c")


@functools.partial(
    pl.kernel,
    out_type=[
        jax.ShapeDtypeStruct((NC * N_PAD,), jnp.float32),
        jax.ShapeDtypeStruct((E_PAD // CHUNK, CHUNK), jnp.int32),
    ],
    mesh=_mesh,
    compiler_params=pltpu.CompilerParams(use_tc_tiling_on_sc=False),
    scratch_types=[
        pltpu.VMEM((EDGES_W,), jnp.int32),
        pltpu.VMEM((EDGES_W,), jnp.int32),
        pltpu.VMEM((NCHUNK, CHUNK), jnp.int32),
        pltpu.VMEM((CHUNK,), jnp.float32),
        pltpu.VMEM((112,), jnp.float32),
        pltpu.VMEM_SHARED((N_PAD,), jnp.float32),
    ],
)
def _sc_degree(row_hbm, col_hbm, deg_out, rowp_out,
               row_v, col_v, rowp_v, ones_v, stage_v, acc):
    c = lax.axis_index("c")
    s = lax.axis_index("s")
    wid = _wid()
    base = wid * EDGES_W
    pltpu.sync_copy(row_hbm.at[pl.ds(base, EDGES_W)], row_v)
    pltpu.sync_copy(col_hbm.at[pl.ds(base, EDGES_W)], col_v)

    def zfill(i, carry):
        stage_v[pl.ds(i * 16, 16)] = jnp.zeros((16,), jnp.float32)
        return carry

    lax.fori_loop(0, 7, zfill, 0)

    def zcopy(i, carry):
        pltpu.sync_copy(stage_v, acc.at[pl.ds(s * STRIPE + i * 112, 112)])
        return carry

    lax.fori_loop(0, STRIPE // 112, zcopy, 0)
    for i in range(CHUNK // 16):
        ones_v[pl.ds(i * 16, 16)] = jnp.full((16,), 1.0, jnp.float32)

    def redirect(j, carry):
        for v in range(CHUNK // 16):
            off = j * CHUNK + v * 16
            r = row_v[pl.ds(off, 16)]
            cc = col_v[pl.ds(off, 16)]
            rowp_v[j, pl.ds(v * 16, 16)] = jnp.where(r == cc, TRASH, r)
        return carry

    lax.fori_loop(0, NCHUNK, redirect, 0)
    plsc.subcore_barrier()

    def scatter(j, carry):
        pltpu.sync_copy(ones_v, acc.at[rowp_v.at[j]], add=True)
        return carry

    lax.fori_loop(0, NCHUNK, scatter, 0)
    pltpu.sync_copy(rowp_v, rowp_out.at[pl.ds(wid * NCHUNK, NCHUNK), :])
    plsc.subcore_barrier()

    def dump(i, carry):
        pltpu.sync_copy(acc.at[pl.ds(s * STRIPE + i * 112, 112)], stage_v)
        pltpu.sync_copy(stage_v,
                        deg_out.at[pl.ds(c * N_PAD + s * STRIPE + i * 112, 112)])
        return carry

    lax.fori_loop(0, STRIPE // 112, dump, 0)


@functools.partial(
    pl.kernel,
    out_type=jax.ShapeDtypeStruct((NC * N_PAD, F_OUT), jnp.float32),
    mesh=_mesh,
    compiler_params=pltpu.CompilerParams(use_tc_tiling_on_sc=False),
    scratch_types=[
        pltpu.VMEM((EDGES_W,), jnp.int32),
        pltpu.VMEM((NCHUNK, CHUNK), jnp.int32),
        pltpu.VMEM((2, CHUNK, F_OUT), jnp.float32),
        pltpu.VMEM((112, F_OUT), jnp.float32),
        pltpu.VMEM_SHARED((N_PAD, F_OUT), jnp.float32),
        pltpu.SemaphoreType.DMA,
    ],
)
def _sc_spmm(tab_hbm, col_hbm, rowp_hbm, acc_out,
             col_v, rowp_v, buf, stage_v, acc, sem):
    c = lax.axis_index("c")
    s = lax.axis_index("s")
    wid = _wid()
    pltpu.sync_copy(col_hbm.at[pl.ds(wid * EDGES_W, EDGES_W)], col_v)
    pltpu.sync_copy(rowp_hbm.at[pl.ds(wid * NCHUNK, NCHUNK), :], rowp_v)

    def zfill(i, carry):
        stage_v[i, pl.ds(0, 16)] = jnp.zeros((16,), jnp.float32)
        return carry

    lax.fori_loop(0, 112, zfill, 0)

    def zcopy(i, carry):
        pltpu.sync_copy(stage_v, acc.at[pl.ds(s * STRIPE + i * 112, 112), :])
        return carry

    lax.fori_loop(0, STRIPE // 112, zcopy, 0)
    plsc.subcore_barrier()

    pltpu.async_copy(tab_hbm.at[col_v.at[pl.ds(0, CHUNK)]], buf.at[0], sem)

    def body(j, carry):
        nxt = j + 1

        @pl.when(nxt < NCHUNK)
        def _():
            pltpu.async_copy(tab_hbm.at[col_v.at[pl.ds(nxt * CHUNK, CHUNK)]],
                             buf.at[nxt % 2], sem)

        pltpu.make_async_copy(tab_hbm.at[col_v.at[pl.ds(j * CHUNK, CHUNK)]],
                              buf.at[j % 2], sem).wait()
        pltpu.sync_copy(buf.at[j % 2], acc.at[rowp_v.at[j]], add=True)
        return carry

    lax.fori_loop(0, NCHUNK, body, 0)
    plsc.subcore_barrier()

    def dump(i, carry):
        pltpu.sync_copy(acc.at[pl.ds(s * STRIPE + i * 112, 112), :], stage_v)
        pltpu.sync_copy(
            stage_v,
            acc_out.at[pl.ds(c * N_PAD + s * STRIPE + i * 112, 112), :])
        return carry

    lax.fori_loop(0, STRIPE // 112, dump, 0)


_BN = 5000


def _tc_a_body(x_ref, w_ref, deg_ref, dis_ref, y0_ref, y1_ref, g1_ref):
    xb = x_ref[...]
    w = w_ref[...]
    d = deg_ref[...]
    deg = d[:, 0:1] + d[:, 1:2]
    dis = jnp.where(deg > 0, lax.rsqrt(deg), 0.0)
    dis_ref[...] = dis
    y0_ref[...] = jnp.dot(xb, w[0] - w[2], preferred_element_type=jnp.float32)
    y1_ref[...] = jnp.dot(xb, w[1], preferred_element_type=jnp.float32)
    y2 = jnp.dot(xb, w[2], preferred_element_type=jnp.float32)
    g1_ref[...] = dis * y2


def _tc_b_body(y1_ref, u0_ref, u1_ref, dis_ref, z_ref):
    dis = dis_ref[...]
    u = u0_ref[...] + u1_ref[...]
    z_ref[...] = dis * y1_ref[...] - 2.0 * (dis * dis) * u


def _tc_c_body(y0_ref, v0_ref, v1_ref, dis_ref, b_ref, o_ref):
    dis = dis_ref[...]
    v = v0_ref[...] + v1_ref[...]
    o_ref[...] = y0_ref[...] - dis * v + b_ref[...]


def _row_spec(width):
    return pl.BlockSpec((_BN, width), lambda i: (i, 0))


_GRID = N_NODES // _BN

_tc_a = pl.pallas_call(
    _tc_a_body,
    grid=(_GRID,),
    in_specs=[
        _row_spec(F_IN),
        pl.BlockSpec((3, F_IN, F_OUT), lambda i: (0, 0, 0)),
        _row_spec(2),
    ],
    out_specs=[_row_spec(1), _row_spec(F_OUT), _row_spec(F_OUT),
               _row_spec(F_OUT)],
    out_shape=[
        jax.ShapeDtypeStruct((N_NODES, 1), jnp.float32),
        jax.ShapeDtypeStruct((N_NODES, F_OUT), jnp.float32),
        jax.ShapeDtypeStruct((N_NODES, F_OUT), jnp.float32),
        jax.ShapeDtypeStruct((N_NODES, F_OUT), jnp.float32),
    ],
)

_tc_b = pl.pallas_call(
    _tc_b_body,
    grid=(_GRID,),
    in_specs=[_row_spec(F_OUT), _row_spec(F_OUT), _row_spec(F_OUT),
              _row_spec(1)],
    out_specs=_row_spec(F_OUT),
    out_shape=jax.ShapeDtypeStruct((N_NODES, F_OUT), jnp.float32),
)

_tc_c = pl.pallas_call(
    _tc_c_body,
    grid=(_GRID,),
    in_specs=[_row_spec(F_OUT), _row_spec(F_OUT), _row_spec(F_OUT),
              _row_spec(1), pl.BlockSpec((1, F_OUT), lambda i: (0, 0))],
    out_specs=_row_spec(F_OUT),
    out_shape=jax.ShapeDtypeStruct((N_NODES, F_OUT), jnp.float32),
)


@jax.jit
def kernel(x, edge_index, weight, bias):
    n = x.shape[0]
    x2 = x.reshape(n, F_IN)
    w = weight.reshape(weight.shape[0], F_IN, F_OUT)

    row = edge_index[0]
    col = edge_index[1]
    pad = E_PAD - E_EDGES
    row_p = jnp.concatenate([row, jnp.full((pad,), TRASH, jnp.int32)])
    col_p = jnp.concatenate([col, jnp.zeros((pad,), jnp.int32)])

    deg2, rowp = _sc_degree(row_p, col_p)
    deg2 = deg2.reshape(NC, N_PAD)
    deg_n = jnp.stack([deg2[0, :n], deg2[1, :n]], axis=1)

    dis, y0, y1, g1 = _tc_a(x2, w, deg_n)

    u = _sc_spmm(g1, col_p, rowp).reshape(NC, N_PAD, F_OUT)
    z = _tc_b(y1, u[0, :n, :], u[1, :n, :], dis)

    v = _sc_spmm(z, col_p, rowp).reshape(NC, N_PAD, F_OUT)
    out_core = _tc_c(y0, v[0, :n, :], v[1, :n, :], dis, bias.reshape(1, F_OUT))

    return out_core.reshape(n, 1, F_OUT, 1)


# speedup vs baseline: 322.5507x; 1.3368x over previous
import functools

import jax
import jax.numpy as jnp
from jax import lax
from jax.experimental import pallas as pl
from jax.experimental.pallas import tpu as pltpu
from jax.experimental.pallas import tpu_sc as plsc

N_NODES = 50000
N_PAD = 50176
TRASH = N_NODES
STRIPE = N_PAD // 16
E_EDGES = 800000
CHUNK = 128
NCHUNK_TOT = E_EDGES // CHUNK
NC, NS = 2, 16
NW = NC * NS
NCH_BASE = NCHUNK_TOT // NW
NCH_XTRA = NCHUNK_TOT - NCH_BASE * NW
MAXCH = NCH_BASE + 1
F_IN = 64
F_OUT = 16

_mesh = plsc.VectorSubcoreMesh(core_axis_name="c", subcore_axis_name="s")


def _tile_work():
    wid = lax.axis_index("s") * NC + lax.axis_index("c")
    base = wid * NCH_BASE + jnp.minimum(wid, NCH_XTRA)
    nch = NCH_BASE + jnp.where(wid < NCH_XTRA, 1, 0)
    return wid, base, nch


@functools.partial(
    pl.kernel,
    out_type=[
        jax.ShapeDtypeStruct((NC * N_PAD,), jnp.float32),
        jax.ShapeDtypeStruct((NCHUNK_TOT, CHUNK), jnp.int32),
    ],
    mesh=_mesh,
    compiler_params=pltpu.CompilerParams(use_tc_tiling_on_sc=False),
    scratch_types=[
        pltpu.VMEM((MAXCH * CHUNK,), jnp.int32),
        pltpu.VMEM((MAXCH * CHUNK,), jnp.int32),
        pltpu.VMEM((MAXCH, CHUNK), jnp.int32),
        pltpu.VMEM((CHUNK,), jnp.float32),
        pltpu.VMEM((112,), jnp.float32),
        pltpu.VMEM_SHARED((N_PAD,), jnp.float32),
        pltpu.SemaphoreType.DMA,
    ],
)
def _sc_degree(row_hbm, col_hbm, deg_out, rowp_out,
               row_v, col_v, rowp_v, ones_v, stage_v, acc, sem):
    c = lax.axis_index("c")
    s = lax.axis_index("s")
    wid, cbase, nch = _tile_work()
    ebase = cbase * CHUNK
    pltpu.sync_copy(row_hbm.at[pl.ds(ebase, NCH_BASE * CHUNK)],
                    row_v.at[pl.ds(0, NCH_BASE * CHUNK)])
    pltpu.sync_copy(col_hbm.at[pl.ds(ebase, NCH_BASE * CHUNK)],
                    col_v.at[pl.ds(0, NCH_BASE * CHUNK)])

    @pl.when(wid < NCH_XTRA)
    def _():
        off = NCH_BASE * CHUNK
        pltpu.sync_copy(row_hbm.at[pl.ds(ebase + off, CHUNK)],
                        row_v.at[pl.ds(off, CHUNK)])
        pltpu.sync_copy(col_hbm.at[pl.ds(ebase + off, CHUNK)],
                        col_v.at[pl.ds(off, CHUNK)])

    def zfill(i, carry):
        stage_v[pl.ds(i * 16, 16)] = jnp.zeros((16,), jnp.float32)
        return carry

    lax.fori_loop(0, 7, zfill, 0)

    def zcopy(i, carry):
        pltpu.sync_copy(stage_v, acc.at[pl.ds(s * STRIPE + i * 112, 112)])
        return carry

    lax.fori_loop(0, STRIPE // 112, zcopy, 0)
    for i in range(CHUNK // 16):
        ones_v[pl.ds(i * 16, 16)] = jnp.full((16,), 1.0, jnp.float32)

    def redirect(j, carry):
        for v in range(CHUNK // 16):
            off = j * CHUNK + v * 16
            r = row_v[pl.ds(off, 16)]
            cc = col_v[pl.ds(off, 16)]
            rowp_v[j, pl.ds(v * 16, 16)] = jnp.where(r == cc, TRASH, r)
        return carry

    lax.fori_loop(0, nch, redirect, 0)
    plsc.subcore_barrier()

    W = 8

    def scatter(j, carry):
        @pl.when(j >= W)
        def _():
            pltpu.make_async_copy(ones_v, acc.at[rowp_v.at[j - W]], sem).wait()

        pltpu.async_copy(ones_v, acc.at[rowp_v.at[j]], sem, add=True)
        return carry

    lax.fori_loop(0, nch, scatter, 0)

    def drain(k, carry):
        pltpu.make_async_copy(ones_v, acc.at[rowp_v.at[nch - W + k]],
                              sem).wait()
        return carry

    lax.fori_loop(0, W, drain, 0)

    pltpu.sync_copy(rowp_v.at[pl.ds(0, NCH_BASE), :],
                    rowp_out.at[pl.ds(cbase, NCH_BASE), :])

    @pl.when(wid < NCH_XTRA)
    def _():
        pltpu.sync_copy(rowp_v.at[pl.ds(NCH_BASE, 1), :],
                        rowp_out.at[pl.ds(cbase + NCH_BASE, 1), :])

    plsc.subcore_barrier()

    def dump(i, carry):
        pltpu.sync_copy(acc.at[pl.ds(s * STRIPE + i * 112, 112)], stage_v)
        pltpu.sync_copy(stage_v,
                        deg_out.at[pl.ds(c * N_PAD + s * STRIPE + i * 112, 112)])
        return carry

    lax.fori_loop(0, STRIPE // 112, dump, 0)


@functools.partial(
    pl.kernel,
    out_type=jax.ShapeDtypeStruct((NC * N_PAD, F_OUT), jnp.float32),
    mesh=_mesh,
    compiler_params=pltpu.CompilerParams(use_tc_tiling_on_sc=False),
    scratch_types=[
        pltpu.VMEM((MAXCH * CHUNK,), jnp.int32),
        pltpu.VMEM((MAXCH, CHUNK), jnp.int32),
        pltpu.VMEM((4, CHUNK, F_OUT), jnp.float32),
        pltpu.VMEM((112, F_OUT), jnp.float32),
        pltpu.VMEM_SHARED((N_PAD, F_OUT), jnp.float32),
        pltpu.SemaphoreType.DMA,
        pltpu.SemaphoreType.DMA,
    ],
)
def _sc_spmm(tab_hbm, col_hbm, rowp_hbm, acc_out,
             col_v, rowp_v, buf, stage_v, acc, semg, sems):
    c = lax.axis_index("c")
    s = lax.axis_index("s")
    wid, cbase, nch = _tile_work()
    ebase = cbase * CHUNK
    pltpu.sync_copy(col_hbm.at[pl.ds(ebase, NCH_BASE * CHUNK)],
                    col_v.at[pl.ds(0, NCH_BASE * CHUNK)])
    pltpu.sync_copy(rowp_hbm.at[pl.ds(cbase, NCH_BASE), :],
                    rowp_v.at[pl.ds(0, NCH_BASE), :])

    @pl.when(wid < NCH_XTRA)
    def _():
        pltpu.sync_copy(col_hbm.at[pl.ds(ebase + NCH_BASE * CHUNK, CHUNK)],
                        col_v.at[pl.ds(NCH_BASE * CHUNK, CHUNK)])
        pltpu.sync_copy(rowp_hbm.at[pl.ds(cbase + NCH_BASE, 1), :],
                        rowp_v.at[pl.ds(NCH_BASE, 1), :])

    def zfill(i, carry):
        stage_v[i, pl.ds(0, 16)] = jnp.zeros((16,), jnp.float32)
        return carry

    lax.fori_loop(0, 112, zfill, 0)

    def zcopy(i, carry):
        pltpu.sync_copy(stage_v, acc.at[pl.ds(s * STRIPE + i * 112, 112), :])
        return carry

    lax.fori_loop(0, STRIPE // 112, zcopy, 0)
    plsc.subcore_barrier()

    def _gather(j, slot):
        pltpu.async_copy(tab_hbm.at[col_v.at[pl.ds(j * CHUNK, CHUNK)]],
                         buf.at[slot], semg)

    def _wait_gather(j, slot):
        pltpu.make_async_copy(tab_hbm.at[col_v.at[pl.ds(j * CHUNK, CHUNK)]],
                              buf.at[slot], semg).wait()

    def _wait_scatter(j, slot):
        pltpu.make_async_copy(buf.at[slot], acc.at[rowp_v.at[j]], sems).wait()

    _gather(0, 0)
    _gather(1, 1)

    def body(j, carry):
        @pl.when(j >= 2)
        def _():
            _wait_scatter(j - 2, (j - 2) % 4)

        @pl.when(j + 2 < nch)
        def _():
            _gather(j + 2, (j + 2) % 4)

        _wait_gather(j, j % 4)
        pltpu.async_copy(buf.at[j % 4], acc.at[rowp_v.at[j]], sems, add=True)
        return carry

    lax.fori_loop(0, nch, body, 0)
    _wait_scatter(nch - 2, (nch - 2) % 4)
    _wait_scatter(nch - 1, (nch - 1) % 4)
    plsc.subcore_barrier()

    def dump(i, carry):
        pltpu.sync_copy(acc.at[pl.ds(s * STRIPE + i * 112, 112), :], stage_v)
        pltpu.sync_copy(
            stage_v,
            acc_out.at[pl.ds(c * N_PAD + s * STRIPE + i * 112, 112), :])
        return carry

    lax.fori_loop(0, STRIPE // 112, dump, 0)


_BN = 5000


def _tc_a_body(x_ref, w_ref, deg_ref, dis_ref, y0_ref, y1_ref, g1_ref):
    xb = x_ref[...]
    w = w_ref[...]
    d = deg_ref[...]
    deg = d[:, 0:1] + d[:, 1:2]
    dis = jnp.where(deg > 0, lax.rsqrt(deg), 0.0)
    dis_ref[...] = dis
    y0_ref[...] = jnp.dot(xb, w[0] - w[2], preferred_element_type=jnp.float32)
    y1_ref[...] = jnp.dot(xb, w[1], preferred_element_type=jnp.float32)
    y2 = jnp.dot(xb, w[2], preferred_element_type=jnp.float32)
    g1_ref[...] = dis * y2


def _tc_b_body(y1_ref, u0_ref, u1_ref, dis_ref, z_ref):
    dis = dis_ref[...]
    u = u0_ref[...] + u1_ref[...]
    z_ref[...] = dis * y1_ref[...] - 2.0 * (dis * dis) * u


def _tc_c_body(y0_ref, v0_ref, v1_ref, dis_ref, b_ref, o_ref):
    dis = dis_ref[...]
    v = v0_ref[...] + v1_ref[...]
    o_ref[...] = y0_ref[...] - dis * v + b_ref[...]


def _row_spec(width):
    return pl.BlockSpec((_BN, width), lambda i: (i, 0))


_GRID = N_NODES // _BN

_tc_a = pl.pallas_call(
    _tc_a_body,
    grid=(_GRID,),
    in_specs=[
        _row_spec(F_IN),
        pl.BlockSpec((3, F_IN, F_OUT), lambda i: (0, 0, 0)),
        _row_spec(2),
    ],
    out_specs=[_row_spec(1), _row_spec(F_OUT), _row_spec(F_OUT),
               _row_spec(F_OUT)],
    out_shape=[
        jax.ShapeDtypeStruct((N_NODES, 1), jnp.float32),
        jax.ShapeDtypeStruct((N_NODES, F_OUT), jnp.float32),
        jax.ShapeDtypeStruct((N_NODES, F_OUT), jnp.float32),
        jax.ShapeDtypeStruct((N_NODES, F_OUT), jnp.float32),
    ],
)

_tc_b = pl.pallas_call(
    _tc_b_body,
    grid=(_GRID,),
    in_specs=[_row_spec(F_OUT), _row_spec(F_OUT), _row_spec(F_OUT),
              _row_spec(1)],
    out_specs=_row_spec(F_OUT),
    out_shape=jax.ShapeDtypeStruct((N_NODES, F_OUT), jnp.float32),
)

_tc_c = pl.pallas_call(
    _tc_c_body,
    grid=(_GRID,),
    in_specs=[_row_spec(F_OUT), _row_spec(F_OUT), _row_spec(F_OUT),
              _row_spec(1), pl.BlockSpec((1, F_OUT), lambda i: (0, 0))],
    out_specs=_row_spec(F_OUT),
    out_shape=jax.ShapeDtypeStruct((N_NODES, F_OUT), jnp.float32),
)


@jax.jit
def kernel(x, edge_index, weight, bias):
    n = x.shape[0]
    x2 = x.reshape(n, F_IN)
    w = weight.reshape(weight.shape[0], F_IN, F_OUT)

    deg2, rowp = _sc_degree(edge_index[0], edge_index[1])
    deg2 = deg2.reshape(NC, N_PAD)
    deg_n = jnp.stack([deg2[0, :n], deg2[1, :n]], axis=1)

    dis, y0, y1, g1 = _tc_a(x2, w, deg_n)

    u = _sc_spmm(g1, edge_index[1], rowp).reshape(NC, N_PAD, F_OUT)
    z = _tc_b(y1, u[0, :n, :], u[1, :n, :], dis)

    v = _sc_spmm(z, edge_index[1], rowp).reshape(NC, N_PAD, F_OUT)
    out_core = _tc_c(y0, v[0, :n, :], v[1, :n, :], dis, bias.reshape(1, F_OUT))

    return out_core.reshape(n, 1, F_OUT, 1)


# speedup vs baseline: 373.8300x; 1.1590x over previous
import functools

import jax
import jax.numpy as jnp
from jax import lax
from jax.experimental import pallas as pl
from jax.experimental.pallas import tpu as pltpu
from jax.experimental.pallas import tpu_sc as plsc

N_NODES = 50000
N_PAD = 50176
TRASH = N_NODES
STRIPE = N_PAD // 16
E_EDGES = 800000
CHUNK = 128
NCHUNK_TOT = E_EDGES // CHUNK
NC, NS = 2, 16
NW = NC * NS
NCH_BASE = NCHUNK_TOT // NW
NCH_XTRA = NCHUNK_TOT - NCH_BASE * NW
MAXCH = NCH_BASE + 1
F_IN = 64
F_OUT = 16

_mesh = plsc.VectorSubcoreMesh(core_axis_name="c", subcore_axis_name="s")


def _tile_work():
    wid = lax.axis_index("s") * NC + lax.axis_index("c")
    base = wid * NCH_BASE + jnp.minimum(wid, NCH_XTRA)
    nch = NCH_BASE + jnp.where(wid < NCH_XTRA, 1, 0)
    return wid, base, nch


@functools.partial(
    pl.kernel,
    out_type=[
        jax.ShapeDtypeStruct((NC * N_PAD,), jnp.float32),
        jax.ShapeDtypeStruct((NCHUNK_TOT, CHUNK), jnp.int32),
    ],
    mesh=_mesh,
    compiler_params=pltpu.CompilerParams(use_tc_tiling_on_sc=False),
    scratch_types=[
        pltpu.VMEM((MAXCH * CHUNK,), jnp.int32),
        pltpu.VMEM((MAXCH * CHUNK,), jnp.int32),
        pltpu.VMEM((MAXCH, CHUNK), jnp.int32),
        pltpu.VMEM((CHUNK,), jnp.float32),
        pltpu.VMEM((112,), jnp.float32),
        pltpu.VMEM_SHARED((N_PAD,), jnp.float32),
        pltpu.SemaphoreType.DMA,
    ],
)
def _sc_degree(ei_hbm, deg_out, rowp_out,
               row_v, col_v, rowp_v, ones_v, stage_v, acc, sem):
    c = lax.axis_index("c")
    s = lax.axis_index("s")
    wid, cbase, nch = _tile_work()
    ebase = cbase * CHUNK
    pltpu.sync_copy(ei_hbm.at[0, pl.ds(ebase, NCH_BASE * CHUNK)],
                    row_v.at[pl.ds(0, NCH_BASE * CHUNK)])
    pltpu.sync_copy(ei_hbm.at[1, pl.ds(ebase, NCH_BASE * CHUNK)],
                    col_v.at[pl.ds(0, NCH_BASE * CHUNK)])

    @pl.when(wid < NCH_XTRA)
    def _():
        off = NCH_BASE * CHUNK
        pltpu.sync_copy(ei_hbm.at[0, pl.ds(ebase + off, CHUNK)],
                        row_v.at[pl.ds(off, CHUNK)])
        pltpu.sync_copy(ei_hbm.at[1, pl.ds(ebase + off, CHUNK)],
                        col_v.at[pl.ds(off, CHUNK)])

    def zfill(i, carry):
        stage_v[pl.ds(i * 16, 16)] = jnp.zeros((16,), jnp.float32)
        return carry

    lax.fori_loop(0, 7, zfill, 0)

    def zcopy(i, carry):
        pltpu.sync_copy(stage_v, acc.at[pl.ds(s * STRIPE + i * 112, 112)])
        return carry

    lax.fori_loop(0, STRIPE // 112, zcopy, 0)
    for i in range(CHUNK // 16):
        ones_v[pl.ds(i * 16, 16)] = jnp.full((16,), 1.0, jnp.float32)

    def redirect(j, carry):
        for v in range(CHUNK // 16):
            off = j * CHUNK + v * 16
            r = row_v[pl.ds(off, 16)]
            cc = col_v[pl.ds(off, 16)]
            rowp_v[j, pl.ds(v * 16, 16)] = jnp.where(r == cc, TRASH, r)
        return carry

    lax.fori_loop(0, nch, redirect, 0)
    plsc.subcore_barrier()

    W = 8

    def scatter(j, carry):
        @pl.when(j >= W)
        def _():
            pltpu.make_async_copy(ones_v, acc.at[rowp_v.at[j - W]], sem).wait()

        pltpu.async_copy(ones_v, acc.at[rowp_v.at[j]], sem, add=True)
        return carry

    lax.fori_loop(0, nch, scatter, 0)

    def drain(k, carry):
        pltpu.make_async_copy(ones_v, acc.at[rowp_v.at[nch - W + k]],
                              sem).wait()
        return carry

    lax.fori_loop(0, W, drain, 0)

    pltpu.sync_copy(rowp_v.at[pl.ds(0, NCH_BASE), :],
                    rowp_out.at[pl.ds(cbase, NCH_BASE), :])

    @pl.when(wid < NCH_XTRA)
    def _():
        pltpu.sync_copy(rowp_v.at[pl.ds(NCH_BASE, 1), :],
                        rowp_out.at[pl.ds(cbase + NCH_BASE, 1), :])

    plsc.subcore_barrier()

    def dump(i, carry):
        pltpu.sync_copy(acc.at[pl.ds(s * STRIPE + i * 112, 112)], stage_v)
        pltpu.sync_copy(stage_v,
                        deg_out.at[pl.ds(c * N_PAD + s * STRIPE + i * 112, 112)])
        return carry

    lax.fori_loop(0, STRIPE // 112, dump, 0)


@functools.partial(
    pl.kernel,
    out_type=[
        jax.ShapeDtypeStruct((N_PAD, F_OUT), jnp.float32),
        jax.ShapeDtypeStruct((N_PAD, F_OUT), jnp.float32),
    ],
    mesh=_mesh,
    compiler_params=pltpu.CompilerParams(use_tc_tiling_on_sc=False),
    scratch_types=[
        pltpu.VMEM((MAXCH * CHUNK,), jnp.int32),
        pltpu.VMEM((MAXCH, CHUNK), jnp.int32),
        pltpu.VMEM((4, CHUNK, F_OUT), jnp.float32),
        pltpu.VMEM((112, F_OUT), jnp.float32),
        pltpu.VMEM_SHARED((N_PAD, F_OUT), jnp.float32),
        pltpu.SemaphoreType.DMA,
        pltpu.SemaphoreType.DMA,
    ],
)
def _sc_spmm(tab_hbm, ei_hbm, rowp_hbm, acc0_out, acc1_out,
             col_v, rowp_v, buf, stage_v, acc, semg, sems):
    c = lax.axis_index("c")
    s = lax.axis_index("s")
    wid, cbase, nch = _tile_work()
    ebase = cbase * CHUNK
    pltpu.sync_copy(ei_hbm.at[1, pl.ds(ebase, NCH_BASE * CHUNK)],
                    col_v.at[pl.ds(0, NCH_BASE * CHUNK)])
    pltpu.sync_copy(rowp_hbm.at[pl.ds(cbase, NCH_BASE), :],
                    rowp_v.at[pl.ds(0, NCH_BASE), :])

    @pl.when(wid < NCH_XTRA)
    def _():
        pltpu.sync_copy(ei_hbm.at[1, pl.ds(ebase + NCH_BASE * CHUNK, CHUNK)],
                        col_v.at[pl.ds(NCH_BASE * CHUNK, CHUNK)])
        pltpu.sync_copy(rowp_hbm.at[pl.ds(cbase + NCH_BASE, 1), :],
                        rowp_v.at[pl.ds(NCH_BASE, 1), :])

    def zfill(i, carry):
        stage_v[i, pl.ds(0, 16)] = jnp.zeros((16,), jnp.float32)
        return carry

    lax.fori_loop(0, 112, zfill, 0)

    def zcopy(i, carry):
        pltpu.sync_copy(stage_v, acc.at[pl.ds(s * STRIPE + i * 112, 112), :])
        return carry

    lax.fori_loop(0, STRIPE // 112, zcopy, 0)
    plsc.subcore_barrier()

    def _gather(j, slot):
        pltpu.async_copy(tab_hbm.at[col_v.at[pl.ds(j * CHUNK, CHUNK)]],
                         buf.at[slot], semg)

    def _wait_gather(j, slot):
        pltpu.make_async_copy(tab_hbm.at[col_v.at[pl.ds(j * CHUNK, CHUNK)]],
                              buf.at[slot], semg).wait()

    def _wait_scatter(j, slot):
        pltpu.make_async_copy(buf.at[slot], acc.at[rowp_v.at[j]], sems).wait()

    _gather(0, 0)
    _gather(1, 1)

    def body(j, carry):
        @pl.when(j >= 2)
        def _():
            _wait_scatter(j - 2, (j - 2) % 4)

        @pl.when(j + 2 < nch)
        def _():
            _gather(j + 2, (j + 2) % 4)

        _wait_gather(j, j % 4)
        pltpu.async_copy(buf.at[j % 4], acc.at[rowp_v.at[j]], sems, add=True)
        return carry

    lax.fori_loop(0, nch, body, 0)
    _wait_scatter(nch - 2, (nch - 2) % 4)
    _wait_scatter(nch - 1, (nch - 1) % 4)
    plsc.subcore_barrier()

    @pl.when(c == 0)
    def _():
        def dump0(i, carry):
            pltpu.sync_copy(acc.at[pl.ds(s * STRIPE + i * 112, 112), :],
                            stage_v)
            pltpu.sync_copy(stage_v,
                            acc0_out.at[pl.ds(s * STRIPE + i * 112, 112), :])
            return carry

        lax.fori_loop(0, STRIPE // 112, dump0, 0)

    @pl.when(c == 1)
    def _():
        def dump1(i, carry):
            pltpu.sync_copy(acc.at[pl.ds(s * STRIPE + i * 112, 112), :],
                            stage_v)
            pltpu.sync_copy(stage_v,
                            acc1_out.at[pl.ds(s * STRIPE + i * 112, 112), :])
            return carry

        lax.fori_loop(0, STRIPE // 112, dump1, 0)


_BN = 5000


def _tc_m_body(x_ref, w_ref, y0_ref, y1_ref, y2_ref):
    xb = x_ref[...]
    w = w_ref[...]
    y0_ref[...] = jnp.dot(xb, w[0] - w[2], preferred_element_type=jnp.float32)
    y1_ref[...] = jnp.dot(xb, w[1], preferred_element_type=jnp.float32)
    y2_ref[...] = jnp.dot(xb, w[2], preferred_element_type=jnp.float32)


def _tc_s_body(deg_ref, y2_ref, dis_ref, g1_ref):
    d = deg_ref[...]
    deg = d[:, 0:1] + d[:, 1:2]
    dis = jnp.where(deg > 0, lax.rsqrt(deg), 0.0)
    dis_ref[...] = dis
    g1_ref[...] = dis * y2_ref[...]


def _tc_b_body(y1_ref, u0_ref, u1_ref, dis_ref, z_ref):
    dis = dis_ref[...]
    u = u0_ref[...] + u1_ref[...]
    z_ref[...] = dis * y1_ref[...] - 2.0 * (dis * dis) * u


def _tc_c_body(y0_ref, v0_ref, v1_ref, dis_ref, b_ref, o_ref):
    dis = dis_ref[...]
    v = v0_ref[...] + v1_ref[...]
    o_ref[...] = y0_ref[...] - dis * v + b_ref[...]


def _row_spec(width):
    return pl.BlockSpec((_BN, width), lambda i: (i, 0))


_GRID = N_NODES // _BN

_tc_m = pl.pallas_call(
    _tc_m_body,
    grid=(_GRID,),
    in_specs=[
        _row_spec(F_IN),
        pl.BlockSpec((3, F_IN, F_OUT), lambda i: (0, 0, 0)),
    ],
    out_specs=[_row_spec(F_OUT), _row_spec(F_OUT), _row_spec(F_OUT)],
    out_shape=[
        jax.ShapeDtypeStruct((N_NODES, F_OUT), jnp.float32),
        jax.ShapeDtypeStruct((N_NODES, F_OUT), jnp.float32),
        jax.ShapeDtypeStruct((N_NODES, F_OUT), jnp.float32),
    ],
)

_tc_s = pl.pallas_call(
    _tc_s_body,
    grid=(_GRID,),
    in_specs=[_row_spec(2), _row_spec(F_OUT)],
    out_specs=[_row_spec(1), _row_spec(F_OUT)],
    out_shape=[
        jax.ShapeDtypeStruct((N_NODES, 1), jnp.float32),
        jax.ShapeDtypeStruct((N_NODES, F_OUT), jnp.float32),
    ],
)

_tc_b = pl.pallas_call(
    _tc_b_body,
    grid=(_GRID,),
    in_specs=[_row_spec(F_OUT), _row_spec(F_OUT), _row_spec(F_OUT),
              _row_spec(1)],
    out_specs=_row_spec(F_OUT),
    out_shape=jax.ShapeDtypeStruct((N_NODES, F_OUT), jnp.float32),
)

_tc_c = pl.pallas_call(
    _tc_c_body,
    grid=(_GRID,),
    in_specs=[_row_spec(F_OUT), _row_spec(F_OUT), _row_spec(F_OUT),
              _row_spec(1), pl.BlockSpec((1, F_OUT), lambda i: (0, 0))],
    out_specs=_row_spec(F_OUT),
    out_shape=jax.ShapeDtypeStruct((N_NODES, F_OUT), jnp.float32),
)


@jax.jit
def kernel(x, edge_index, weight, bias):
    n = x.shape[0]
    x2 = x.reshape(n, F_IN)
    w = weight.reshape(weight.shape[0], F_IN, F_OUT)

    deg2, rowp = _sc_degree(edge_index)
    deg2 = deg2.reshape(NC, N_PAD)
    deg_n = jnp.stack([deg2[0, :n], deg2[1, :n]], axis=1)

    y0, y1, y2 = _tc_m(x2, w)
    dis, g1 = _tc_s(deg_n, y2)

    u0, u1 = _sc_spmm(g1, edge_index, rowp)
    z = _tc_b(y1, u0, u1, dis)

    v0, v1 = _sc_spmm(z, edge_index, rowp)
    out_core = _tc_c(y0, v0, v1, dis, bias.reshape(1, F_OUT))

    return out_core.reshape(n, 1, F_OUT, 1)


# speedup vs baseline: 471.3894x; 1.2610x over previous
import functools

import jax
import jax.numpy as jnp
from jax import lax
from jax.experimental import pallas as pl
from jax.experimental.pallas import tpu as pltpu
from jax.experimental.pallas import tpu_sc as plsc

N_NODES = 50000
N_PAD = 50176
TRASH = N_NODES
STRIPE = N_PAD // 16
E_EDGES = 800000
CHUNK = 128
NCHUNK_TOT = E_EDGES // CHUNK
NC, NS = 2, 16
NW = NC * NS
NCH_BASE = NCHUNK_TOT // NW
NCH_XTRA = NCHUNK_TOT - NCH_BASE * NW
MAXCH = NCH_BASE + 1
F_IN = 64
F_OUT = 16

_mesh = plsc.VectorSubcoreMesh(core_axis_name="c", subcore_axis_name="s")


def _tile_work():
    wid = lax.axis_index("s") * NC + lax.axis_index("c")
    base = wid * NCH_BASE + jnp.minimum(wid, NCH_XTRA)
    nch = NCH_BASE + jnp.where(wid < NCH_XTRA, 1, 0)
    return wid, base, nch


@functools.partial(
    pl.kernel,
    out_type=[
        jax.ShapeDtypeStruct((N_PAD,), jnp.float32),
        jax.ShapeDtypeStruct((N_PAD,), jnp.float32),
        jax.ShapeDtypeStruct((NCHUNK_TOT, CHUNK), jnp.int32),
    ],
    mesh=_mesh,
    compiler_params=pltpu.CompilerParams(use_tc_tiling_on_sc=False),
    scratch_types=[
        pltpu.VMEM((MAXCH * CHUNK,), jnp.int32),
        pltpu.VMEM((MAXCH * CHUNK,), jnp.int32),
        pltpu.VMEM((MAXCH, CHUNK), jnp.int32),
        pltpu.VMEM((CHUNK,), jnp.float32),
        pltpu.VMEM((112,), jnp.float32),
        pltpu.VMEM_SHARED((N_PAD,), jnp.float32),
        pltpu.SemaphoreType.DMA,
    ],
)
def _sc_degree(ei_hbm, deg0_out, deg1_out, rowp_out,
               row_v, col_v, rowp_v, ones_v, stage_v, acc, sem):
    c = lax.axis_index("c")
    s = lax.axis_index("s")
    wid, cbase, nch = _tile_work()
    ebase = cbase * CHUNK
    pltpu.sync_copy(ei_hbm.at[0, pl.ds(ebase, NCH_BASE * CHUNK)],
                    row_v.at[pl.ds(0, NCH_BASE * CHUNK)])
    pltpu.sync_copy(ei_hbm.at[1, pl.ds(ebase, NCH_BASE * CHUNK)],
                    col_v.at[pl.ds(0, NCH_BASE * CHUNK)])

    @pl.when(wid < NCH_XTRA)
    def _():
        off = NCH_BASE * CHUNK
        pltpu.sync_copy(ei_hbm.at[0, pl.ds(ebase + off, CHUNK)],
                        row_v.at[pl.ds(off, CHUNK)])
        pltpu.sync_copy(ei_hbm.at[1, pl.ds(ebase + off, CHUNK)],
                        col_v.at[pl.ds(off, CHUNK)])

    def zfill(i, carry):
        stage_v[pl.ds(i * 16, 16)] = jnp.zeros((16,), jnp.float32)
        return carry

    lax.fori_loop(0, 7, zfill, 0)

    def zcopy(i, carry):
        pltpu.sync_copy(stage_v, acc.at[pl.ds(s * STRIPE + i * 112, 112)])
        return carry

    lax.fori_loop(0, STRIPE // 112, zcopy, 0)
    for i in range(CHUNK // 16):
        ones_v[pl.ds(i * 16, 16)] = jnp.full((16,), 1.0, jnp.float32)

    def redirect(j, carry):
        for v in range(CHUNK // 16):
            off = j * CHUNK + v * 16
            r = row_v[pl.ds(off, 16)]
            cc = col_v[pl.ds(off, 16)]
            rowp_v[j, pl.ds(v * 16, 16)] = jnp.where(r == cc, TRASH, r)
        return carry

    lax.fori_loop(0, nch, redirect, 0)
    plsc.subcore_barrier()

    W = 8

    def scatter(j, carry):
        @pl.when(j >= W)
        def _():
            pltpu.make_async_copy(ones_v, acc.at[rowp_v.at[j - W]], sem).wait()

        pltpu.async_copy(ones_v, acc.at[rowp_v.at[j]], sem, add=True)
        return carry

    lax.fori_loop(0, nch, scatter, 0)

    def drain(k, carry):
        pltpu.make_async_copy(ones_v, acc.at[rowp_v.at[nch - W + k]],
                              sem).wait()
        return carry

    lax.fori_loop(0, W, drain, 0)

    pltpu.sync_copy(rowp_v.at[pl.ds(0, NCH_BASE), :],
                    rowp_out.at[pl.ds(cbase, NCH_BASE), :])

    @pl.when(wid < NCH_XTRA)
    def _():
        pltpu.sync_copy(rowp_v.at[pl.ds(NCH_BASE, 1), :],
                        rowp_out.at[pl.ds(cbase + NCH_BASE, 1), :])

    plsc.subcore_barrier()

    @pl.when(c == 0)
    def _():
        def dump0(i, carry):
            pltpu.sync_copy(acc.at[pl.ds(s * STRIPE + i * 112, 112)], stage_v)
            pltpu.sync_copy(stage_v,
                            deg0_out.at[pl.ds(s * STRIPE + i * 112, 112)])
            return carry

        lax.fori_loop(0, STRIPE // 112, dump0, 0)

    @pl.when(c == 1)
    def _():
        def dump1(i, carry):
            pltpu.sync_copy(acc.at[pl.ds(s * STRIPE + i * 112, 112)], stage_v)
            pltpu.sync_copy(stage_v,
                            deg1_out.at[pl.ds(s * STRIPE + i * 112, 112)])
            return carry

        lax.fori_loop(0, STRIPE // 112, dump1, 0)


@functools.partial(
    pl.kernel,
    out_type=[
        jax.ShapeDtypeStruct((N_PAD, F_OUT), jnp.float32),
        jax.ShapeDtypeStruct((N_PAD, F_OUT), jnp.float32),
    ],
    mesh=_mesh,
    compiler_params=pltpu.CompilerParams(use_tc_tiling_on_sc=False),
    scratch_types=[
        pltpu.VMEM((MAXCH * CHUNK,), jnp.int32),
        pltpu.VMEM((MAXCH, CHUNK), jnp.int32),
        pltpu.VMEM((4, CHUNK, F_OUT), jnp.float32),
        pltpu.VMEM((112, F_OUT), jnp.float32),
        pltpu.VMEM_SHARED((N_PAD, F_OUT), jnp.float32),
        pltpu.SemaphoreType.DMA,
        pltpu.SemaphoreType.DMA,
    ],
)
def _sc_spmm(tab_hbm, ei_hbm, rowp_hbm, acc0_out, acc1_out,
             col_v, rowp_v, buf, stage_v, acc, semg, sems):
    c = lax.axis_index("c")
    s = lax.axis_index("s")
    wid, cbase, nch = _tile_work()
    ebase = cbase * CHUNK
    pltpu.sync_copy(ei_hbm.at[1, pl.ds(ebase, NCH_BASE * CHUNK)],
                    col_v.at[pl.ds(0, NCH_BASE * CHUNK)])
    pltpu.sync_copy(rowp_hbm.at[pl.ds(cbase, NCH_BASE), :],
                    rowp_v.at[pl.ds(0, NCH_BASE), :])

    @pl.when(wid < NCH_XTRA)
    def _():
        pltpu.sync_copy(ei_hbm.at[1, pl.ds(ebase + NCH_BASE * CHUNK, CHUNK)],
                        col_v.at[pl.ds(NCH_BASE * CHUNK, CHUNK)])
        pltpu.sync_copy(rowp_hbm.at[pl.ds(cbase + NCH_BASE, 1), :],
                        rowp_v.at[pl.ds(NCH_BASE, 1), :])

    def zfill(i, carry):
        stage_v[i, pl.ds(0, 16)] = jnp.zeros((16,), jnp.float32)
        return carry

    lax.fori_loop(0, 112, zfill, 0)

    def zcopy(i, carry):
        pltpu.sync_copy(stage_v, acc.at[pl.ds(s * STRIPE + i * 112, 112), :])
        return carry

    lax.fori_loop(0, STRIPE // 112, zcopy, 0)
    plsc.subcore_barrier()

    def _gather(j, slot):
        pltpu.async_copy(tab_hbm.at[col_v.at[pl.ds(j * CHUNK, CHUNK)]],
                         buf.at[slot], semg)

    def _wait_gather(j, slot):
        pltpu.make_async_copy(tab_hbm.at[col_v.at[pl.ds(j * CHUNK, CHUNK)]],
                              buf.at[slot], semg).wait()

    def _wait_scatter(j, slot):
        pltpu.make_async_copy(buf.at[slot], acc.at[rowp_v.at[j]], sems).wait()

    _gather(0, 0)
    _gather(1, 1)

    def body(j, carry):
        @pl.when(j >= 2)
        def _():
            _wait_scatter(j - 2, (j - 2) % 4)

        @pl.when(j + 2 < nch)
        def _():
            _gather(j + 2, (j + 2) % 4)

        _wait_gather(j, j % 4)
        pltpu.async_copy(buf.at[j % 4], acc.at[rowp_v.at[j]], sems, add=True)
        return carry

    lax.fori_loop(0, nch, body, 0)
    _wait_scatter(nch - 2, (nch - 2) % 4)
    _wait_scatter(nch - 1, (nch - 1) % 4)
    plsc.subcore_barrier()

    @pl.when(c == 0)
    def _():
        def dump0(i, carry):
            pltpu.sync_copy(acc.at[pl.ds(s * STRIPE + i * 112, 112), :],
                            stage_v)
            pltpu.sync_copy(stage_v,
                            acc0_out.at[pl.ds(s * STRIPE + i * 112, 112), :])
            return carry

        lax.fori_loop(0, STRIPE // 112, dump0, 0)

    @pl.when(c == 1)
    def _():
        def dump1(i, carry):
            pltpu.sync_copy(acc.at[pl.ds(s * STRIPE + i * 112, 112), :],
                            stage_v)
            pltpu.sync_copy(stage_v,
                            acc1_out.at[pl.ds(s * STRIPE + i * 112, 112), :])
            return carry

        lax.fori_loop(0, STRIPE // 112, dump1, 0)


_BN = 5000


NR = N_PAD // 8
NRX = N_NODES // 8
_RB = NR // 8
_GRID = 8


def _rep_mat():
    return jnp.repeat(jnp.eye(8, dtype=jnp.float32), F_OUT, axis=1)


def _dis128(d0_ref, d1_ref):
    deg = d0_ref[...] + d1_ref[...]
    dis8 = jnp.where(deg > 0, lax.rsqrt(deg), 0.0)
    return jnp.dot(dis8, _rep_mat(), preferred_element_type=jnp.float32)


def _tc_m_body(x_ref, w_ref, y0_ref, y1_ref, y2_ref):
    xb = x_ref[...]
    w = w_ref[...]
    y0_ref[...] = jnp.dot(xb, w[0], preferred_element_type=jnp.float32)
    y1_ref[...] = jnp.dot(xb, w[1], preferred_element_type=jnp.float32)
    y2_ref[...] = jnp.dot(xb, w[2], preferred_element_type=jnp.float32)


def _tc_s_body(d0_ref, d1_ref, y2_ref, g1_ref):
    g1_ref[...] = _dis128(d0_ref, d1_ref) * y2_ref[...]


def _tc_b_body(y1_ref, u0_ref, u1_ref, d0_ref, d1_ref, z_ref):
    dis = _dis128(d0_ref, d1_ref)
    u = u0_ref[...] + u1_ref[...]
    z_ref[...] = dis * y1_ref[...] - 2.0 * (dis * dis) * u


def _tc_c_body(y0_ref, v0_ref, v1_ref, d0_ref, d1_ref, b_ref, o_ref):
    dis = _dis128(d0_ref, d1_ref)
    v = v0_ref[...] + v1_ref[...]
    o_ref[...] = y0_ref[...] - dis * v + b_ref[...]


def _p_spec(width):
    return pl.BlockSpec((_RB, width), lambda i: (i, 0))


_PK = jax.ShapeDtypeStruct((NR, 128), jnp.float32)

_tc_m = pl.pallas_call(
    _tc_m_body,
    grid=(_GRID,),
    in_specs=[
        _p_spec(8 * F_IN),
        pl.BlockSpec((3, 8 * F_IN, 128), lambda i: (0, 0, 0)),
    ],
    out_specs=[_p_spec(128), _p_spec(128), _p_spec(128)],
    out_shape=[_PK, _PK, _PK],
)

_tc_s = pl.pallas_call(
    _tc_s_body,
    grid=(_GRID,),
    in_specs=[_p_spec(8), _p_spec(8), _p_spec(128)],
    out_specs=_p_spec(128),
    out_shape=_PK,
)

_tc_b = pl.pallas_call(
    _tc_b_body,
    grid=(_GRID,),
    in_specs=[_p_spec(128), _p_spec(128), _p_spec(128), _p_spec(8),
              _p_spec(8)],
    out_specs=_p_spec(128),
    out_shape=_PK,
)

_tc_c = pl.pallas_call(
    _tc_c_body,
    grid=(_GRID,),
    in_specs=[_p_spec(128), _p_spec(128), _p_spec(128), _p_spec(8),
              _p_spec(8), pl.BlockSpec((1, 128), lambda i: (0, 0))],
    out_specs=_p_spec(128),
    out_shape=_PK,
)


@jax.jit
def kernel(x, edge_index, weight, bias):
    n = x.shape[0]
    x_p = x.reshape(NRX, 8 * F_IN)
    w = weight.reshape(weight.shape[0], F_IN, F_OUT)
    eye8 = jnp.eye(8, dtype=jnp.float32)
    wbd = jnp.stack([jnp.kron(eye8, w[0] - w[2]),
                     jnp.kron(eye8, w[1]),
                     jnp.kron(eye8, w[2])])

    deg0, deg1, rowp = _sc_degree(edge_index)
    d0_8 = deg0.reshape(NR, 8)
    d1_8 = deg1.reshape(NR, 8)

    y0, y1, y2 = _tc_m(x_p, wbd)
    g1 = _tc_s(d0_8, d1_8, y2)

    u0, u1 = _sc_spmm(g1.reshape(N_PAD, F_OUT), edge_index, rowp)
    z = _tc_b(y1, u0.reshape(NR, 128), u1.reshape(NR, 128), d0_8, d1_8)

    v0, v1 = _sc_spmm(z.reshape(N_PAD, F_OUT), edge_index, rowp)
    out_p = _tc_c(y0, v0.reshape(NR, 128), v1.reshape(NR, 128), d0_8, d1_8,
                  jnp.tile(bias, 8).reshape(1, 128))

    return out_p.reshape(N_PAD, F_OUT)[:n].reshape(n, 1, F_OUT, 1)


# speedup vs baseline: 531.9913x; 1.1286x over previous
import functools

import jax
import jax.numpy as jnp
from jax import lax
from jax.experimental import pallas as pl
from jax.experimental.pallas import tpu as pltpu
from jax.experimental.pallas import tpu_sc as plsc

N_NODES = 50000
N_PAD = 50176
TRASH = N_NODES
STRIPE = N_PAD // 16
E_EDGES = 800000
CHUNK = 128
NCHUNK_TOT = E_EDGES // CHUNK
NC, NS = 2, 16
NW = NC * NS
NCH_BASE = NCHUNK_TOT // NW
NCH_XTRA = NCHUNK_TOT - NCH_BASE * NW
MAXCH = NCH_BASE + 1
F_IN = 64
F_OUT = 16

_mesh = plsc.VectorSubcoreMesh(core_axis_name="c", subcore_axis_name="s")


def _tile_work():
    wid = lax.axis_index("s") * NC + lax.axis_index("c")
    base = wid * NCH_BASE + jnp.minimum(wid, NCH_XTRA)
    nch = NCH_BASE + jnp.where(wid < NCH_XTRA, 1, 0)
    return wid, base, nch


@functools.partial(
    pl.kernel,
    out_type=[
        jax.ShapeDtypeStruct((N_PAD,), jnp.float32),
        jax.ShapeDtypeStruct((N_PAD,), jnp.float32),
        jax.ShapeDtypeStruct((NCHUNK_TOT, CHUNK), jnp.int32),
    ],
    mesh=_mesh,
    compiler_params=pltpu.CompilerParams(use_tc_tiling_on_sc=False),
    scratch_types=[
        pltpu.VMEM((MAXCH * CHUNK,), jnp.int32),
        pltpu.VMEM((MAXCH * CHUNK,), jnp.int32),
        pltpu.VMEM((MAXCH, CHUNK), jnp.int32),
        pltpu.VMEM((CHUNK,), jnp.float32),
        pltpu.VMEM((112,), jnp.float32),
        pltpu.VMEM_SHARED((N_PAD,), jnp.float32),
        pltpu.SemaphoreType.DMA,
    ],
)
def _sc_degree(ei_hbm, deg0_out, deg1_out, rowp_out,
               row_v, col_v, rowp_v, ones_v, stage_v, acc, sem):
    c = lax.axis_index("c")
    s = lax.axis_index("s")
    wid, cbase, nch = _tile_work()
    ebase = cbase * CHUNK
    pltpu.sync_copy(ei_hbm.at[0, pl.ds(ebase, NCH_BASE * CHUNK)],
                    row_v.at[pl.ds(0, NCH_BASE * CHUNK)])
    pltpu.sync_copy(ei_hbm.at[1, pl.ds(ebase, NCH_BASE * CHUNK)],
                    col_v.at[pl.ds(0, NCH_BASE * CHUNK)])

    @pl.when(wid < NCH_XTRA)
    def _():
        off = NCH_BASE * CHUNK
        pltpu.sync_copy(ei_hbm.at[0, pl.ds(ebase + off, CHUNK)],
                        row_v.at[pl.ds(off, CHUNK)])
        pltpu.sync_copy(ei_hbm.at[1, pl.ds(ebase + off, CHUNK)],
                        col_v.at[pl.ds(off, CHUNK)])

    def zfill(i, carry):
        stage_v[pl.ds(i * 16, 16)] = jnp.zeros((16,), jnp.float32)
        return carry

    lax.fori_loop(0, 7, zfill, 0)

    def zcopy(i, carry):
        pltpu.sync_copy(stage_v, acc.at[pl.ds(s * STRIPE + i * 112, 112)])
        return carry

    lax.fori_loop(0, STRIPE // 112, zcopy, 0)
    for i in range(CHUNK // 16):
        ones_v[pl.ds(i * 16, 16)] = jnp.full((16,), 1.0, jnp.float32)

    def redirect(j, carry):
        for v in range(CHUNK // 16):
            off = j * CHUNK + v * 16
            r = row_v[pl.ds(off, 16)]
            cc = col_v[pl.ds(off, 16)]
            rowp_v[j, pl.ds(v * 16, 16)] = jnp.where(r == cc, TRASH, r)
        return carry

    lax.fori_loop(0, nch, redirect, 0)
    plsc.subcore_barrier()

    W = 8

    def scatter(j, carry):
        @pl.when(j >= W)
        def _():
            pltpu.make_async_copy(ones_v, acc.at[rowp_v.at[j - W]], sem).wait()

        pltpu.async_copy(ones_v, acc.at[rowp_v.at[j]], sem, add=True)
        return carry

    lax.fori_loop(0, nch, scatter, 0)

    def drain(k, carry):
        pltpu.make_async_copy(ones_v, acc.at[rowp_v.at[nch - W + k]],
                              sem).wait()
        return carry

    lax.fori_loop(0, W, drain, 0)

    pltpu.sync_copy(rowp_v.at[pl.ds(0, NCH_BASE), :],
                    rowp_out.at[pl.ds(cbase, NCH_BASE), :])

    @pl.when(wid < NCH_XTRA)
    def _():
        pltpu.sync_copy(rowp_v.at[pl.ds(NCH_BASE, 1), :],
                        rowp_out.at[pl.ds(cbase + NCH_BASE, 1), :])

    plsc.subcore_barrier()

    @pl.when(c == 0)
    def _():
        def dump0(i, carry):
            pltpu.sync_copy(acc.at[pl.ds(s * STRIPE + i * 112, 112)], stage_v)
            pltpu.sync_copy(stage_v,
                            deg0_out.at[pl.ds(s * STRIPE + i * 112, 112)])
            return carry

        lax.fori_loop(0, STRIPE // 112, dump0, 0)

    @pl.when(c == 1)
    def _():
        def dump1(i, carry):
            pltpu.sync_copy(acc.at[pl.ds(s * STRIPE + i * 112, 112)], stage_v)
            pltpu.sync_copy(stage_v,
                            deg1_out.at[pl.ds(s * STRIPE + i * 112, 112)])
            return carry

        lax.fori_loop(0, STRIPE // 112, dump1, 0)


@functools.partial(
    pl.kernel,
    out_type=[
        jax.ShapeDtypeStruct((N_PAD, F_OUT), jnp.float32),
        jax.ShapeDtypeStruct((N_PAD, F_OUT), jnp.float32),
    ],
    mesh=_mesh,
    compiler_params=pltpu.CompilerParams(use_tc_tiling_on_sc=False),
    scratch_types=[
        pltpu.VMEM((MAXCH * CHUNK,), jnp.int32),
        pltpu.VMEM((MAXCH, CHUNK), jnp.int32),
        pltpu.VMEM((8, CHUNK, F_OUT), jnp.float32),
        pltpu.VMEM((112, F_OUT), jnp.float32),
        pltpu.VMEM_SHARED((N_PAD, F_OUT), jnp.float32),
        pltpu.SemaphoreType.DMA,
        pltpu.SemaphoreType.DMA,
    ],
)
def _sc_spmm(tab_hbm, ei_hbm, rowp_hbm, acc0_out, acc1_out,
             col_v, rowp_v, buf, stage_v, acc, semg, sems):
    c = lax.axis_index("c")
    s = lax.axis_index("s")
    wid, cbase, nch = _tile_work()
    ebase = cbase * CHUNK
    pltpu.sync_copy(ei_hbm.at[1, pl.ds(ebase, NCH_BASE * CHUNK)],
                    col_v.at[pl.ds(0, NCH_BASE * CHUNK)])
    pltpu.sync_copy(rowp_hbm.at[pl.ds(cbase, NCH_BASE), :],
                    rowp_v.at[pl.ds(0, NCH_BASE), :])

    @pl.when(wid < NCH_XTRA)
    def _():
        pltpu.sync_copy(ei_hbm.at[1, pl.ds(ebase + NCH_BASE * CHUNK, CHUNK)],
                        col_v.at[pl.ds(NCH_BASE * CHUNK, CHUNK)])
        pltpu.sync_copy(rowp_hbm.at[pl.ds(cbase + NCH_BASE, 1), :],
                        rowp_v.at[pl.ds(NCH_BASE, 1), :])

    def zfill(i, carry):
        stage_v[i, pl.ds(0, 16)] = jnp.zeros((16,), jnp.float32)
        return carry

    lax.fori_loop(0, 112, zfill, 0)

    def zcopy(i, carry):
        pltpu.sync_copy(stage_v, acc.at[pl.ds(s * STRIPE + i * 112, 112), :])
        return carry

    lax.fori_loop(0, STRIPE // 112, zcopy, 0)
    plsc.subcore_barrier()

    def _gather(j, slot):
        pltpu.async_copy(tab_hbm.at[col_v.at[pl.ds(j * CHUNK, CHUNK)]],
                         buf.at[slot], semg)

    def _wait_gather(j, slot):
        pltpu.make_async_copy(tab_hbm.at[col_v.at[pl.ds(j * CHUNK, CHUNK)]],
                              buf.at[slot], semg).wait()

    def _wait_scatter(j, slot):
        pltpu.make_async_copy(buf.at[slot], acc.at[rowp_v.at[j]], sems).wait()

    for k in range(4):
        _gather(k, k)

    def body(j, carry):
        @pl.when(j >= 4)
        def _():
            _wait_scatter(j - 4, (j - 4) % 8)

        @pl.when(j + 4 < nch)
        def _():
            _gather(j + 4, (j + 4) % 8)

        _wait_gather(j, j % 8)
        pltpu.async_copy(buf.at[j % 8], acc.at[rowp_v.at[j]], sems, add=True)
        return carry

    lax.fori_loop(0, nch, body, 0)
    for k in range(4):
        _wait_scatter(nch - 4 + k, (nch - 4 + k) % 8)
    plsc.subcore_barrier()

    @pl.when(c == 0)
    def _():
        def dump0(i, carry):
            pltpu.sync_copy(acc.at[pl.ds(s * STRIPE + i * 112, 112), :],
                            stage_v)
            pltpu.sync_copy(stage_v,
                            acc0_out.at[pl.ds(s * STRIPE + i * 112, 112), :])
            return carry

        lax.fori_loop(0, STRIPE // 112, dump0, 0)

    @pl.when(c == 1)
    def _():
        def dump1(i, carry):
            pltpu.sync_copy(acc.at[pl.ds(s * STRIPE + i * 112, 112), :],
                            stage_v)
            pltpu.sync_copy(stage_v,
                            acc1_out.at[pl.ds(s * STRIPE + i * 112, 112), :])
            return carry

        lax.fori_loop(0, STRIPE // 112, dump1, 0)


_BN = 5000


NR = N_PAD // 8
NRX = N_NODES // 8
_RB = NR // 8
_GRID = 8


def _rep_mat():
    return jnp.repeat(jnp.eye(8, dtype=jnp.float32), F_OUT, axis=1)


def _dis128(d0_ref, d1_ref):
    deg = d0_ref[...] + d1_ref[...]
    dis8 = jnp.where(deg > 0, lax.rsqrt(deg), 0.0)
    return jnp.dot(dis8, _rep_mat(), preferred_element_type=jnp.float32)


def _tc_m_body(x_ref, w_ref, y0_ref, y1_ref, y2_ref):
    xb = x_ref[...]
    w = w_ref[...]
    y0_ref[...] = jnp.dot(xb, w[0], preferred_element_type=jnp.float32)
    y1_ref[...] = jnp.dot(xb, w[1], preferred_element_type=jnp.float32)
    y2_ref[...] = jnp.dot(xb, w[2], preferred_element_type=jnp.float32)


def _tc_s_body(d0_ref, d1_ref, y2_ref, g1_ref):
    g1_ref[...] = _dis128(d0_ref, d1_ref) * y2_ref[...]


def _tc_b_body(y1_ref, u0_ref, u1_ref, d0_ref, d1_ref, z_ref):
    dis = _dis128(d0_ref, d1_ref)
    u = u0_ref[...] + u1_ref[...]
    z_ref[...] = dis * y1_ref[...] - 2.0 * (dis * dis) * u


def _tc_c_body(y0_ref, v0_ref, v1_ref, d0_ref, d1_ref, b_ref, o_ref):
    dis = _dis128(d0_ref, d1_ref)
    v = v0_ref[...] + v1_ref[...]
    o_ref[...] = y0_ref[...] - dis * v + b_ref[...]


def _p_spec(width):
    return pl.BlockSpec((_RB, width), lambda i: (i, 0))


_PK = jax.ShapeDtypeStruct((NR, 128), jnp.float32)

_tc_m = pl.pallas_call(
    _tc_m_body,
    grid=(_GRID,),
    in_specs=[
        _p_spec(8 * F_IN),
        pl.BlockSpec((3, 8 * F_IN, 128), lambda i: (0, 0, 0)),
    ],
    out_specs=[_p_spec(128), _p_spec(128), _p_spec(128)],
    out_shape=[_PK, _PK, _PK],
)

_tc_s = pl.pallas_call(
    _tc_s_body,
    grid=(_GRID,),
    in_specs=[_p_spec(8), _p_spec(8), _p_spec(128)],
    out_specs=_p_spec(128),
    out_shape=_PK,
)

_tc_b = pl.pallas_call(
    _tc_b_body,
    grid=(_GRID,),
    in_specs=[_p_spec(128), _p_spec(128), _p_spec(128), _p_spec(8),
              _p_spec(8)],
    out_specs=_p_spec(128),
    out_shape=_PK,
)

_tc_c = pl.pallas_call(
    _tc_c_body,
    grid=(_GRID,),
    in_specs=[_p_spec(128), _p_spec(128), _p_spec(128), _p_spec(8),
              _p_spec(8), pl.BlockSpec((1, 128), lambda i: (0, 0))],
    out_specs=_p_spec(128),
    out_shape=_PK,
)


@jax.jit
def kernel(x, edge_index, weight, bias):
    n = x.shape[0]
    x_p = x.reshape(NRX, 8 * F_IN)
    w = weight.reshape(weight.shape[0], F_IN, F_OUT)
    eye8 = jnp.eye(8, dtype=jnp.float32)
    wbd = jnp.stack([jnp.kron(eye8, w[0] - w[2]),
                     jnp.kron(eye8, w[1]),
                     jnp.kron(eye8, w[2])])

    deg0, deg1, rowp = _sc_degree(edge_index)
    d0_8 = deg0.reshape(NR, 8)
    d1_8 = deg1.reshape(NR, 8)

    y0, y1, y2 = _tc_m(x_p, wbd)
    g1 = _tc_s(d0_8, d1_8, y2)

    u0, u1 = _sc_spmm(g1.reshape(N_PAD, F_OUT), edge_index, rowp)
    z = _tc_b(y1, u0.reshape(NR, 128), u1.reshape(NR, 128), d0_8, d1_8)

    v0, v1 = _sc_spmm(z.reshape(N_PAD, F_OUT), edge_index, rowp)
    out_p = _tc_c(y0, v0.reshape(NR, 128), v1.reshape(NR, 128), d0_8, d1_8,
                  jnp.tile(bias, 8).reshape(1, 128))

    return out_p.reshape(N_PAD, F_OUT)[:n].reshape(n, 1, F_OUT, 1)


# speedup vs baseline: 566.9009x; 1.0656x over previous
import functools

import jax
import jax.numpy as jnp
from jax import lax
from jax.experimental import pallas as pl
from jax.experimental.pallas import tpu as pltpu
from jax.experimental.pallas import tpu_sc as plsc

N_NODES = 50000
N_PAD = 50176
TRASH = N_NODES
STRIPE = N_PAD // 16
E_EDGES = 800000
CHUNK = 128
NCHUNK_TOT = E_EDGES // CHUNK
NC, NS = 2, 16
NW = NC * NS
NCH_BASE = NCHUNK_TOT // NW
NCH_XTRA = NCHUNK_TOT - NCH_BASE * NW
MAXCH = NCH_BASE + 1
F_IN = 64
F_OUT = 16

_mesh = plsc.VectorSubcoreMesh(core_axis_name="c", subcore_axis_name="s")


def _tile_work():
    wid = lax.axis_index("s") * NC + lax.axis_index("c")
    base = wid * NCH_BASE + jnp.minimum(wid, NCH_XTRA)
    nch = NCH_BASE + jnp.where(wid < NCH_XTRA, 1, 0)
    return wid, base, nch


@functools.partial(
    pl.kernel,
    out_type=[
        jax.ShapeDtypeStruct((N_PAD,), jnp.float32),
        jax.ShapeDtypeStruct((N_PAD,), jnp.float32),
        jax.ShapeDtypeStruct((NCHUNK_TOT, CHUNK), jnp.int32),
    ],
    mesh=_mesh,
    compiler_params=pltpu.CompilerParams(use_tc_tiling_on_sc=False),
    scratch_types=[
        pltpu.VMEM((MAXCH * CHUNK,), jnp.int32),
        pltpu.VMEM((MAXCH * CHUNK,), jnp.int32),
        pltpu.VMEM((MAXCH, CHUNK), jnp.int32),
        pltpu.VMEM((CHUNK,), jnp.float32),
        pltpu.VMEM((112,), jnp.float32),
        pltpu.VMEM_SHARED((N_PAD,), jnp.float32),
        pltpu.SemaphoreType.DMA,
    ],
)
def _sc_degree(ei_hbm, deg0_out, deg1_out, rowp_out,
               row_v, col_v, rowp_v, ones_v, stage_v, acc, sem):
    c = lax.axis_index("c")
    s = lax.axis_index("s")
    wid, cbase, nch = _tile_work()
    ebase = cbase * CHUNK
    pltpu.sync_copy(ei_hbm.at[0, pl.ds(ebase, NCH_BASE * CHUNK)],
                    row_v.at[pl.ds(0, NCH_BASE * CHUNK)])
    pltpu.sync_copy(ei_hbm.at[1, pl.ds(ebase, NCH_BASE * CHUNK)],
                    col_v.at[pl.ds(0, NCH_BASE * CHUNK)])

    @pl.when(wid < NCH_XTRA)
    def _():
        off = NCH_BASE * CHUNK
        pltpu.sync_copy(ei_hbm.at[0, pl.ds(ebase + off, CHUNK)],
                        row_v.at[pl.ds(off, CHUNK)])
        pltpu.sync_copy(ei_hbm.at[1, pl.ds(ebase + off, CHUNK)],
                        col_v.at[pl.ds(off, CHUNK)])

    def zfill(i, carry):
        stage_v[pl.ds(i * 16, 16)] = jnp.zeros((16,), jnp.float32)
        return carry

    lax.fori_loop(0, 7, zfill, 0)

    def zcopy(i, carry):
        pltpu.sync_copy(stage_v, acc.at[pl.ds(s * STRIPE + i * 112, 112)])
        return carry

    lax.fori_loop(0, STRIPE // 112, zcopy, 0)
    for i in range(CHUNK // 16):
        ones_v[pl.ds(i * 16, 16)] = jnp.full((16,), 1.0, jnp.float32)

    def redirect(j, carry):
        for v in range(CHUNK // 16):
            off = j * CHUNK + v * 16
            r = row_v[pl.ds(off, 16)]
            cc = col_v[pl.ds(off, 16)]
            rowp_v[j, pl.ds(v * 16, 16)] = jnp.where(r == cc, TRASH, r)
        return carry

    lax.fori_loop(0, nch, redirect, 0)
    plsc.subcore_barrier()

    W = 8

    def scatter(j, carry):
        @pl.when(j >= W)
        def _():
            pltpu.make_async_copy(ones_v, acc.at[rowp_v.at[j - W]], sem).wait()

        pltpu.async_copy(ones_v, acc.at[rowp_v.at[j]], sem, add=True)
        return carry

    lax.fori_loop(0, nch, scatter, 0)

    def drain(k, carry):
        pltpu.make_async_copy(ones_v, acc.at[rowp_v.at[nch - W + k]],
                              sem).wait()
        return carry

    lax.fori_loop(0, W, drain, 0)

    pltpu.sync_copy(rowp_v.at[pl.ds(0, NCH_BASE), :],
                    rowp_out.at[pl.ds(cbase, NCH_BASE), :])

    @pl.when(wid < NCH_XTRA)
    def _():
        pltpu.sync_copy(rowp_v.at[pl.ds(NCH_BASE, 1), :],
                        rowp_out.at[pl.ds(cbase + NCH_BASE, 1), :])

    plsc.subcore_barrier()

    @pl.when(c == 0)
    def _():
        def dump0(i, carry):
            pltpu.sync_copy(acc.at[pl.ds(s * STRIPE + i * 112, 112)], stage_v)
            pltpu.sync_copy(stage_v,
                            deg0_out.at[pl.ds(s * STRIPE + i * 112, 112)])
            return carry

        lax.fori_loop(0, STRIPE // 112, dump0, 0)

    @pl.when(c == 1)
    def _():
        def dump1(i, carry):
            pltpu.sync_copy(acc.at[pl.ds(s * STRIPE + i * 112, 112)], stage_v)
            pltpu.sync_copy(stage_v,
                            deg1_out.at[pl.ds(s * STRIPE + i * 112, 112)])
            return carry

        lax.fori_loop(0, STRIPE // 112, dump1, 0)


@functools.partial(
    pl.kernel,
    out_type=[
        jax.ShapeDtypeStruct((N_PAD, F_OUT), jnp.float32),
        jax.ShapeDtypeStruct((N_PAD, F_OUT), jnp.float32),
    ],
    mesh=_mesh,
    compiler_params=pltpu.CompilerParams(use_tc_tiling_on_sc=False),
    scratch_types=[
        pltpu.VMEM((MAXCH * CHUNK,), jnp.int32),
        pltpu.VMEM((MAXCH, CHUNK), jnp.int32),
        pltpu.VMEM((8, CHUNK, F_OUT), jnp.float32),
        pltpu.VMEM((112, F_OUT), jnp.float32),
        pltpu.VMEM_SHARED((N_PAD, F_OUT), jnp.float32),
        pltpu.SemaphoreType.DMA,
        pltpu.SemaphoreType.DMA,
    ],
)
def _sc_spmm(tab_hbm, ei_hbm, rowp_hbm, acc0_out, acc1_out,
             col_v, rowp_v, buf, stage_v, acc, semg, sems):
    c = lax.axis_index("c")
    s = lax.axis_index("s")
    wid, cbase, nch = _tile_work()
    ebase = cbase * CHUNK
    pltpu.sync_copy(ei_hbm.at[1, pl.ds(ebase, NCH_BASE * CHUNK)],
                    col_v.at[pl.ds(0, NCH_BASE * CHUNK)])
    pltpu.sync_copy(rowp_hbm.at[pl.ds(cbase, NCH_BASE), :],
                    rowp_v.at[pl.ds(0, NCH_BASE), :])

    @pl.when(wid < NCH_XTRA)
    def _():
        pltpu.sync_copy(ei_hbm.at[1, pl.ds(ebase + NCH_BASE * CHUNK, CHUNK)],
                        col_v.at[pl.ds(NCH_BASE * CHUNK, CHUNK)])
        pltpu.sync_copy(rowp_hbm.at[pl.ds(cbase + NCH_BASE, 1), :],
                        rowp_v.at[pl.ds(NCH_BASE, 1), :])

    def zfill(i, carry):
        stage_v[i, pl.ds(0, 16)] = jnp.zeros((16,), jnp.float32)
        return carry

    lax.fori_loop(0, 112, zfill, 0)

    def zcopy(i, carry):
        pltpu.sync_copy(stage_v, acc.at[pl.ds(s * STRIPE + i * 112, 112), :])
        return carry

    lax.fori_loop(0, STRIPE // 112, zcopy, 0)
    plsc.subcore_barrier()

    def _gather(j, slot):
        pltpu.async_copy(tab_hbm.at[col_v.at[pl.ds(j * CHUNK, CHUNK)]],
                         buf.at[slot], semg)

    def _wait_gather(j, slot):
        pltpu.make_async_copy(tab_hbm.at[col_v.at[pl.ds(j * CHUNK, CHUNK)]],
                              buf.at[slot], semg).wait()

    def _wait_scatter(j, slot):
        pltpu.make_async_copy(buf.at[slot], acc.at[rowp_v.at[j]], sems).wait()

    for k in range(6):
        _gather(k, k)

    def body(j, carry):
        @pl.when(j >= 2)
        def _():
            _wait_scatter(j - 2, (j - 2) % 8)

        @pl.when(j + 6 < nch)
        def _():
            _gather(j + 6, (j + 6) % 8)

        _wait_gather(j, j % 8)
        pltpu.async_copy(buf.at[j % 8], acc.at[rowp_v.at[j]], sems, add=True)
        return carry

    lax.fori_loop(0, nch, body, 0)
    for k in range(2):
        _wait_scatter(nch - 2 + k, (nch - 2 + k) % 8)
    plsc.subcore_barrier()

    @pl.when(c == 0)
    def _():
        def dump0(i, carry):
            pltpu.sync_copy(acc.at[pl.ds(s * STRIPE + i * 112, 112), :],
                            stage_v)
            pltpu.sync_copy(stage_v,
                            acc0_out.at[pl.ds(s * STRIPE + i * 112, 112), :])
            return carry

        lax.fori_loop(0, STRIPE // 112, dump0, 0)

    @pl.when(c == 1)
    def _():
        def dump1(i, carry):
            pltpu.sync_copy(acc.at[pl.ds(s * STRIPE + i * 112, 112), :],
                            stage_v)
            pltpu.sync_copy(stage_v,
                            acc1_out.at[pl.ds(s * STRIPE + i * 112, 112), :])
            return carry

        lax.fori_loop(0, STRIPE // 112, dump1, 0)


_BN = 5000


NR = N_PAD // 8
NRX = N_NODES // 8
_RB = NR // 8
_GRID = 8


def _rep_mat():
    return jnp.repeat(jnp.eye(8, dtype=jnp.float32), F_OUT, axis=1)


def _dis128(d0_ref, d1_ref):
    deg = d0_ref[...] + d1_ref[...]
    dis8 = jnp.where(deg > 0, lax.rsqrt(deg), 0.0)
    return jnp.dot(dis8, _rep_mat(), preferred_element_type=jnp.float32)


def _tc_m_body(x_ref, w_ref, y0_ref, y1_ref, y2_ref):
    xb = x_ref[...]
    w = w_ref[...]
    y0_ref[...] = jnp.dot(xb, w[0], preferred_element_type=jnp.float32)
    y1_ref[...] = jnp.dot(xb, w[1], preferred_element_type=jnp.float32)
    y2_ref[...] = jnp.dot(xb, w[2], preferred_element_type=jnp.float32)


def _tc_s_body(d0_ref, d1_ref, y2_ref, g1_ref):
    g1_ref[...] = _dis128(d0_ref, d1_ref) * y2_ref[...]


def _tc_b_body(y1_ref, u0_ref, u1_ref, d0_ref, d1_ref, z_ref):
    dis = _dis128(d0_ref, d1_ref)
    u = u0_ref[...] + u1_ref[...]
    z_ref[...] = dis * y1_ref[...] - 2.0 * (dis * dis) * u


def _tc_c_body(y0_ref, v0_ref, v1_ref, d0_ref, d1_ref, b_ref, o_ref):
    dis = _dis128(d0_ref, d1_ref)
    v = v0_ref[...] + v1_ref[...]
    o_ref[...] = y0_ref[...] - dis * v + b_ref[...]


def _p_spec(width):
    return pl.BlockSpec((_RB, width), lambda i: (i, 0))


_PK = jax.ShapeDtypeStruct((NR, 128), jnp.float32)

_tc_m = pl.pallas_call(
    _tc_m_body,
    grid=(_GRID,),
    in_specs=[
        _p_spec(8 * F_IN),
        pl.BlockSpec((3, 8 * F_IN, 128), lambda i: (0, 0, 0)),
    ],
    out_specs=[_p_spec(128), _p_spec(128), _p_spec(128)],
    out_shape=[_PK, _PK, _PK],
)

_tc_s = pl.pallas_call(
    _tc_s_body,
    grid=(_GRID,),
    in_specs=[_p_spec(8), _p_spec(8), _p_spec(128)],
    out_specs=_p_spec(128),
    out_shape=_PK,
)

_tc_b = pl.pallas_call(
    _tc_b_body,
    grid=(_GRID,),
    in_specs=[_p_spec(128), _p_spec(128), _p_spec(128), _p_spec(8),
              _p_spec(8)],
    out_specs=_p_spec(128),
    out_shape=_PK,
)

_tc_c = pl.pallas_call(
    _tc_c_body,
    grid=(_GRID,),
    in_specs=[_p_spec(128), _p_spec(128), _p_spec(128), _p_spec(8),
              _p_spec(8), pl.BlockSpec((1, 128), lambda i: (0, 0))],
    out_specs=_p_spec(128),
    out_shape=_PK,
)


@jax.jit
def kernel(x, edge_index, weight, bias):
    n = x.shape[0]
    x_p = x.reshape(NRX, 8 * F_IN)
    w = weight.reshape(weight.shape[0], F_IN, F_OUT)
    eye8 = jnp.eye(8, dtype=jnp.float32)
    wbd = jnp.stack([jnp.kron(eye8, w[0] - w[2]),
                     jnp.kron(eye8, w[1]),
                     jnp.kron(eye8, w[2])])

    deg0, deg1, rowp = _sc_degree(edge_index)
    d0_8 = deg0.reshape(NR, 8)
    d1_8 = deg1.reshape(NR, 8)

    y0, y1, y2 = _tc_m(x_p, wbd)
    g1 = _tc_s(d0_8, d1_8, y2)

    u0, u1 = _sc_spmm(g1.reshape(N_PAD, F_OUT), edge_index, rowp)
    z = _tc_b(y1, u0.reshape(NR, 128), u1.reshape(NR, 128), d0_8, d1_8)

    v0, v1 = _sc_spmm(z.reshape(N_PAD, F_OUT), edge_index, rowp)
    out_p = _tc_c(y0, v0.reshape(NR, 128), v1.reshape(NR, 128), d0_8, d1_8,
                  jnp.tile(bias, 8).reshape(1, 128))

    return out_p.reshape(N_PAD, F_OUT)[:n].reshape(n, 1, F_OUT, 1)


# speedup vs baseline: 589.9591x; 1.0407x over previous
import functools

import jax
import jax.numpy as jnp
from jax import lax
from jax.experimental import pallas as pl
from jax.experimental.pallas import tpu as pltpu
from jax.experimental.pallas import tpu_sc as plsc

N_NODES = 50000
N_PAD = 50176
TRASH = N_NODES
STRIPE = N_PAD // 16
E_EDGES = 800000
CHUNK = 128
NCHUNK_TOT = E_EDGES // CHUNK
NC, NS = 2, 16
NW = NC * NS
NCH_BASE = NCHUNK_TOT // NW
NCH_XTRA = NCHUNK_TOT - NCH_BASE * NW
MAXCH = NCH_BASE + 1
F_IN = 64
F_OUT = 16

_mesh = plsc.VectorSubcoreMesh(core_axis_name="c", subcore_axis_name="s")


def _tile_work():
    wid = lax.axis_index("s") * NC + lax.axis_index("c")
    base = wid * NCH_BASE + jnp.minimum(wid, NCH_XTRA)
    nch = NCH_BASE + jnp.where(wid < NCH_XTRA, 1, 0)
    return wid, base, nch


@functools.partial(
    pl.kernel,
    out_type=[
        jax.ShapeDtypeStruct((N_PAD,), jnp.float32),
        jax.ShapeDtypeStruct((N_PAD,), jnp.float32),
        jax.ShapeDtypeStruct((NCHUNK_TOT, CHUNK), jnp.int32),
    ],
    mesh=_mesh,
    compiler_params=pltpu.CompilerParams(use_tc_tiling_on_sc=False),
    scratch_types=[
        pltpu.VMEM((MAXCH * CHUNK,), jnp.int32),
        pltpu.VMEM((MAXCH * CHUNK,), jnp.int32),
        pltpu.VMEM((MAXCH, CHUNK), jnp.int32),
        pltpu.VMEM((CHUNK,), jnp.float32),
        pltpu.VMEM((112,), jnp.float32),
        pltpu.VMEM_SHARED((N_PAD,), jnp.float32),
        pltpu.SemaphoreType.DMA,
    ],
)
def _sc_degree(ei_hbm, deg0_out, deg1_out, rowp_out,
               row_v, col_v, rowp_v, ones_v, stage_v, acc, sem):
    c = lax.axis_index("c")
    s = lax.axis_index("s")
    wid, cbase, nch = _tile_work()
    ebase = cbase * CHUNK
    pltpu.sync_copy(ei_hbm.at[0, pl.ds(ebase, NCH_BASE * CHUNK)],
                    row_v.at[pl.ds(0, NCH_BASE * CHUNK)])
    pltpu.sync_copy(ei_hbm.at[1, pl.ds(ebase, NCH_BASE * CHUNK)],
                    col_v.at[pl.ds(0, NCH_BASE * CHUNK)])

    @pl.when(wid < NCH_XTRA)
    def _():
        off = NCH_BASE * CHUNK
        pltpu.sync_copy(ei_hbm.at[0, pl.ds(ebase + off, CHUNK)],
                        row_v.at[pl.ds(off, CHUNK)])
        pltpu.sync_copy(ei_hbm.at[1, pl.ds(ebase + off, CHUNK)],
                        col_v.at[pl.ds(off, CHUNK)])

    def zfill(i, carry):
        stage_v[pl.ds(i * 16, 16)] = jnp.zeros((16,), jnp.float32)
        return carry

    lax.fori_loop(0, 7, zfill, 0)

    def zcopy(i, carry):
        pltpu.sync_copy(stage_v, acc.at[pl.ds(s * STRIPE + i * 112, 112)])
        return carry

    lax.fori_loop(0, STRIPE // 112, zcopy, 0)
    for i in range(CHUNK // 16):
        ones_v[pl.ds(i * 16, 16)] = jnp.full((16,), 1.0, jnp.float32)

    def redirect(j, carry):
        for v in range(CHUNK // 16):
            off = j * CHUNK + v * 16
            r = row_v[pl.ds(off, 16)]
            cc = col_v[pl.ds(off, 16)]
            rowp_v[j, pl.ds(v * 16, 16)] = jnp.where(r == cc, TRASH, r)
        return carry

    lax.fori_loop(0, nch, redirect, 0)
    plsc.subcore_barrier()

    W = 8

    def scatter(j, carry):
        @pl.when(j >= W)
        def _():
            pltpu.make_async_copy(ones_v, acc.at[rowp_v.at[j - W]], sem).wait()

        pltpu.async_copy(ones_v, acc.at[rowp_v.at[j]], sem, add=True)
        return carry

    lax.fori_loop(0, nch, scatter, 0)

    def drain(k, carry):
        pltpu.make_async_copy(ones_v, acc.at[rowp_v.at[nch - W + k]],
                              sem).wait()
        return carry

    lax.fori_loop(0, W, drain, 0)

    pltpu.sync_copy(rowp_v.at[pl.ds(0, NCH_BASE), :],
                    rowp_out.at[pl.ds(cbase, NCH_BASE), :])

    @pl.when(wid < NCH_XTRA)
    def _():
        pltpu.sync_copy(rowp_v.at[pl.ds(NCH_BASE, 1), :],
                        rowp_out.at[pl.ds(cbase + NCH_BASE, 1), :])

    plsc.subcore_barrier()

    @pl.when(c == 0)
    def _():
        def dump0(i, carry):
            pltpu.sync_copy(acc.at[pl.ds(s * STRIPE + i * 112, 112)], stage_v)
            pltpu.sync_copy(stage_v,
                            deg0_out.at[pl.ds(s * STRIPE + i * 112, 112)])
            return carry

        lax.fori_loop(0, STRIPE // 112, dump0, 0)

    @pl.when(c == 1)
    def _():
        def dump1(i, carry):
            pltpu.sync_copy(acc.at[pl.ds(s * STRIPE + i * 112, 112)], stage_v)
            pltpu.sync_copy(stage_v,
                            deg1_out.at[pl.ds(s * STRIPE + i * 112, 112)])
            return carry

        lax.fori_loop(0, STRIPE // 112, dump1, 0)


@functools.partial(
    pl.kernel,
    out_type=[
        jax.ShapeDtypeStruct((N_PAD, F_OUT), jnp.float32),
        jax.ShapeDtypeStruct((N_PAD, F_OUT), jnp.float32),
    ],
    mesh=_mesh,
    compiler_params=pltpu.CompilerParams(use_tc_tiling_on_sc=False),
    scratch_types=[
        pltpu.VMEM((MAXCH * CHUNK,), jnp.int32),
        pltpu.VMEM((MAXCH, CHUNK), jnp.int32),
        pltpu.VMEM((12, CHUNK, F_OUT), jnp.float32),
        pltpu.VMEM((112, F_OUT), jnp.float32),
        pltpu.VMEM_SHARED((N_PAD, F_OUT), jnp.float32),
        pltpu.SemaphoreType.DMA,
        pltpu.SemaphoreType.DMA,
    ],
)
def _sc_spmm(tab_hbm, ei_hbm, rowp_hbm, acc0_out, acc1_out,
             col_v, rowp_v, buf, stage_v, acc, semg, sems):
    c = lax.axis_index("c")
    s = lax.axis_index("s")
    wid, cbase, nch = _tile_work()
    ebase = cbase * CHUNK
    pltpu.sync_copy(ei_hbm.at[1, pl.ds(ebase, NCH_BASE * CHUNK)],
                    col_v.at[pl.ds(0, NCH_BASE * CHUNK)])
    pltpu.sync_copy(rowp_hbm.at[pl.ds(cbase, NCH_BASE), :],
                    rowp_v.at[pl.ds(0, NCH_BASE), :])

    @pl.when(wid < NCH_XTRA)
    def _():
        pltpu.sync_copy(ei_hbm.at[1, pl.ds(ebase + NCH_BASE * CHUNK, CHUNK)],
                        col_v.at[pl.ds(NCH_BASE * CHUNK, CHUNK)])
        pltpu.sync_copy(rowp_hbm.at[pl.ds(cbase + NCH_BASE, 1), :],
                        rowp_v.at[pl.ds(NCH_BASE, 1), :])

    def zfill(i, carry):
        stage_v[i, pl.ds(0, 16)] = jnp.zeros((16,), jnp.float32)
        return carry

    lax.fori_loop(0, 112, zfill, 0)

    def zcopy(i, carry):
        pltpu.sync_copy(stage_v, acc.at[pl.ds(s * STRIPE + i * 112, 112), :])
        return carry

    lax.fori_loop(0, STRIPE // 112, zcopy, 0)
    plsc.subcore_barrier()

    def _gather(j, slot):
        pltpu.async_copy(tab_hbm.at[col_v.at[pl.ds(j * CHUNK, CHUNK)]],
                         buf.at[slot], semg)

    def _wait_gather(j, slot):
        pltpu.make_async_copy(tab_hbm.at[col_v.at[pl.ds(j * CHUNK, CHUNK)]],
                              buf.at[slot], semg).wait()

    def _wait_scatter(j, slot):
        pltpu.make_async_copy(buf.at[slot], acc.at[rowp_v.at[j]], sems).wait()

    for k in range(8):
        _gather(k, k)

    def body(j, carry):
        @pl.when(j >= 4)
        def _():
            _wait_scatter(j - 4, (j - 4) % 12)

        @pl.when(j + 8 < nch)
        def _():
            _gather(j + 8, (j + 8) % 12)

        _wait_gather(j, j % 12)
        pltpu.async_copy(buf.at[j % 12], acc.at[rowp_v.at[j]], sems, add=True)
        return carry

    lax.fori_loop(0, nch, body, 0)
    for k in range(4):
        _wait_scatter(nch - 4 + k, (nch - 4 + k) % 12)
    plsc.subcore_barrier()

    @pl.when(c == 0)
    def _():
        def dump0(i, carry):
            pltpu.sync_copy(acc.at[pl.ds(s * STRIPE + i * 112, 112), :],
                            stage_v)
            pltpu.sync_copy(stage_v,
                            acc0_out.at[pl.ds(s * STRIPE + i * 112, 112), :])
            return carry

        lax.fori_loop(0, STRIPE // 112, dump0, 0)

    @pl.when(c == 1)
    def _():
        def dump1(i, carry):
            pltpu.sync_copy(acc.at[pl.ds(s * STRIPE + i * 112, 112), :],
                            stage_v)
            pltpu.sync_copy(stage_v,
                            acc1_out.at[pl.ds(s * STRIPE + i * 112, 112), :])
            return carry

        lax.fori_loop(0, STRIPE // 112, dump1, 0)


_BN = 5000


NR = N_PAD // 8
NRX = N_NODES // 8
_RB = NR // 8
_GRID = 8


def _rep_mat():
    return jnp.repeat(jnp.eye(8, dtype=jnp.float32), F_OUT, axis=1)


def _dis128(d0_ref, d1_ref):
    deg = d0_ref[...] + d1_ref[...]
    dis8 = jnp.where(deg > 0, lax.rsqrt(deg), 0.0)
    return jnp.dot(dis8, _rep_mat(), preferred_element_type=jnp.float32)


def _tc_m_body(x_ref, w_ref, y0_ref, y1_ref, y2_ref):
    xb = x_ref[...]
    w = w_ref[...]
    y0_ref[...] = jnp.dot(xb, w[0], preferred_element_type=jnp.float32)
    y1_ref[...] = jnp.dot(xb, w[1], preferred_element_type=jnp.float32)
    y2_ref[...] = jnp.dot(xb, w[2], preferred_element_type=jnp.float32)


def _tc_s_body(d0_ref, d1_ref, y2_ref, g1_ref):
    g1_ref[...] = _dis128(d0_ref, d1_ref) * y2_ref[...]


def _tc_b_body(y1_ref, u0_ref, u1_ref, d0_ref, d1_ref, z_ref):
    dis = _dis128(d0_ref, d1_ref)
    u = u0_ref[...] + u1_ref[...]
    z_ref[...] = dis * y1_ref[...] - 2.0 * (dis * dis) * u


def _tc_c_body(y0_ref, v0_ref, v1_ref, d0_ref, d1_ref, b_ref, o_ref):
    dis = _dis128(d0_ref, d1_ref)
    v = v0_ref[...] + v1_ref[...]
    o_ref[...] = y0_ref[...] - dis * v + b_ref[...]


def _p_spec(width):
    return pl.BlockSpec((_RB, width), lambda i: (i, 0))


_PK = jax.ShapeDtypeStruct((NR, 128), jnp.float32)

_tc_m = pl.pallas_call(
    _tc_m_body,
    grid=(_GRID,),
    in_specs=[
        _p_spec(8 * F_IN),
        pl.BlockSpec((3, 8 * F_IN, 128), lambda i: (0, 0, 0)),
    ],
    out_specs=[_p_spec(128), _p_spec(128), _p_spec(128)],
    out_shape=[_PK, _PK, _PK],
)

_tc_s = pl.pallas_call(
    _tc_s_body,
    grid=(_GRID,),
    in_specs=[_p_spec(8), _p_spec(8), _p_spec(128)],
    out_specs=_p_spec(128),
    out_shape=_PK,
)

_tc_b = pl.pallas_call(
    _tc_b_body,
    grid=(_GRID,),
    in_specs=[_p_spec(128), _p_spec(128), _p_spec(128), _p_spec(8),
              _p_spec(8)],
    out_specs=_p_spec(128),
    out_shape=_PK,
)

_tc_c = pl.pallas_call(
    _tc_c_body,
    grid=(_GRID,),
    in_specs=[_p_spec(128), _p_spec(128), _p_spec(128), _p_spec(8),
              _p_spec(8), pl.BlockSpec((1, 128), lambda i: (0, 0))],
    out_specs=_p_spec(128),
    out_shape=_PK,
)


@jax.jit
def kernel(x, edge_index, weight, bias):
    n = x.shape[0]
    x_p = x.reshape(NRX, 8 * F_IN)
    w = weight.reshape(weight.shape[0], F_IN, F_OUT)
    eye8 = jnp.eye(8, dtype=jnp.float32)
    wbd = jnp.stack([jnp.kron(eye8, w[0] - w[2]),
                     jnp.kron(eye8, w[1]),
                     jnp.kron(eye8, w[2])])

    deg0, deg1, rowp = _sc_degree(edge_index)
    d0_8 = deg0.reshape(NR, 8)
    d1_8 = deg1.reshape(NR, 8)

    y0, y1, y2 = _tc_m(x_p, wbd)
    g1 = _tc_s(d0_8, d1_8, y2)

    u0, u1 = _sc_spmm(g1.reshape(N_PAD, F_OUT), edge_index, rowp)
    z = _tc_b(y1, u0.reshape(NR, 128), u1.reshape(NR, 128), d0_8, d1_8)

    v0, v1 = _sc_spmm(z.reshape(N_PAD, F_OUT), edge_index, rowp)
    out_p = _tc_c(y0, v0.reshape(NR, 128), v1.reshape(NR, 128), d0_8, d1_8,
                  jnp.tile(bias, 8).reshape(1, 128))

    return out_p.reshape(N_PAD, F_OUT)[:n].reshape(n, 1, F_OUT, 1)


# speedup vs baseline: 598.7003x; 1.0148x over previous
import functools

import jax
import jax.numpy as jnp
from jax import lax
from jax.experimental import pallas as pl
from jax.experimental.pallas import tpu as pltpu
from jax.experimental.pallas import tpu_sc as plsc

N_NODES = 50000
N_PAD = 50176
TRASH = N_NODES
STRIPE = N_PAD // 16
E_EDGES = 800000
CHUNK = 128
NCHUNK_TOT = E_EDGES // CHUNK
NC, NS = 2, 16
NW = NC * NS
NCH_BASE = NCHUNK_TOT // NW
NCH_XTRA = NCHUNK_TOT - NCH_BASE * NW
MAXCH = NCH_BASE + 1
F_IN = 64
F_OUT = 16

_mesh = plsc.VectorSubcoreMesh(core_axis_name="c", subcore_axis_name="s")


def _tile_work():
    wid = lax.axis_index("s") * NC + lax.axis_index("c")
    base = wid * NCH_BASE + jnp.minimum(wid, NCH_XTRA)
    nch = NCH_BASE + jnp.where(wid < NCH_XTRA, 1, 0)
    return wid, base, nch


@functools.partial(
    pl.kernel,
    out_type=[
        jax.ShapeDtypeStruct((N_PAD,), jnp.float32),
        jax.ShapeDtypeStruct((N_PAD,), jnp.float32),
        jax.ShapeDtypeStruct((NCHUNK_TOT, CHUNK), jnp.int32),
    ],
    mesh=_mesh,
    compiler_params=pltpu.CompilerParams(use_tc_tiling_on_sc=False),
    scratch_types=[
        pltpu.VMEM((MAXCH * CHUNK,), jnp.int32),
        pltpu.VMEM((MAXCH * CHUNK,), jnp.int32),
        pltpu.VMEM((MAXCH, CHUNK), jnp.int32),
        pltpu.VMEM((CHUNK,), jnp.float32),
        pltpu.VMEM((112,), jnp.float32),
        pltpu.VMEM_SHARED((N_PAD,), jnp.float32),
        pltpu.SemaphoreType.DMA,
    ],
)
def _sc_degree(ei_hbm, deg0_out, deg1_out, rowp_out,
               row_v, col_v, rowp_v, ones_v, stage_v, acc, sem):
    c = lax.axis_index("c")
    s = lax.axis_index("s")
    wid, cbase, nch = _tile_work()
    ebase = cbase * CHUNK
    pltpu.sync_copy(ei_hbm.at[0, pl.ds(ebase, NCH_BASE * CHUNK)],
                    row_v.at[pl.ds(0, NCH_BASE * CHUNK)])
    pltpu.sync_copy(ei_hbm.at[1, pl.ds(ebase, NCH_BASE * CHUNK)],
                    col_v.at[pl.ds(0, NCH_BASE * CHUNK)])

    @pl.when(wid < NCH_XTRA)
    def _():
        off = NCH_BASE * CHUNK
        pltpu.sync_copy(ei_hbm.at[0, pl.ds(ebase + off, CHUNK)],
                        row_v.at[pl.ds(off, CHUNK)])
        pltpu.sync_copy(ei_hbm.at[1, pl.ds(ebase + off, CHUNK)],
                        col_v.at[pl.ds(off, CHUNK)])

    def zfill(i, carry):
        stage_v[pl.ds(i * 16, 16)] = jnp.zeros((16,), jnp.float32)
        return carry

    lax.fori_loop(0, 7, zfill, 0)

    def zcopy(i, carry):
        pltpu.sync_copy(stage_v, acc.at[pl.ds(s * STRIPE + i * 112, 112)])
        return carry

    lax.fori_loop(0, STRIPE // 112, zcopy, 0)
    for i in range(CHUNK // 16):
        ones_v[pl.ds(i * 16, 16)] = jnp.full((16,), 1.0, jnp.float32)

    def redirect(j, carry):
        for v in range(CHUNK // 16):
            off = j * CHUNK + v * 16
            r = row_v[pl.ds(off, 16)]
            cc = col_v[pl.ds(off, 16)]
            rowp_v[j, pl.ds(v * 16, 16)] = jnp.where(r == cc, TRASH, r)
        return carry

    lax.fori_loop(0, nch, redirect, 0)
    plsc.subcore_barrier()

    W = 8

    def scatter(j, carry):
        @pl.when(j >= W)
        def _():
            pltpu.make_async_copy(ones_v, acc.at[rowp_v.at[j - W]], sem).wait()

        pltpu.async_copy(ones_v, acc.at[rowp_v.at[j]], sem, add=True)
        return carry

    lax.fori_loop(0, nch, scatter, 0)

    def drain(k, carry):
        pltpu.make_async_copy(ones_v, acc.at[rowp_v.at[nch - W + k]],
                              sem).wait()
        return carry

    lax.fori_loop(0, W, drain, 0)

    pltpu.sync_copy(rowp_v.at[pl.ds(0, NCH_BASE), :],
                    rowp_out.at[pl.ds(cbase, NCH_BASE), :])

    @pl.when(wid < NCH_XTRA)
    def _():
        pltpu.sync_copy(rowp_v.at[pl.ds(NCH_BASE, 1), :],
                        rowp_out.at[pl.ds(cbase + NCH_BASE, 1), :])

    plsc.subcore_barrier()

    @pl.when(c == 0)
    def _():
        def dump0(i, carry):
            pltpu.sync_copy(acc.at[pl.ds(s * STRIPE + i * 112, 112)], stage_v)
            pltpu.sync_copy(stage_v,
                            deg0_out.at[pl.ds(s * STRIPE + i * 112, 112)])
            return carry

        lax.fori_loop(0, STRIPE // 112, dump0, 0)

    @pl.when(c == 1)
    def _():
        def dump1(i, carry):
            pltpu.sync_copy(acc.at[pl.ds(s * STRIPE + i * 112, 112)], stage_v)
            pltpu.sync_copy(stage_v,
                            deg1_out.at[pl.ds(s * STRIPE + i * 112, 112)])
            return carry

        lax.fori_loop(0, STRIPE // 112, dump1, 0)


@functools.partial(
    pl.kernel,
    out_type=[
        jax.ShapeDtypeStruct((N_PAD, F_OUT), jnp.float32),
        jax.ShapeDtypeStruct((N_PAD, F_OUT), jnp.float32),
    ],
    mesh=_mesh,
    compiler_params=pltpu.CompilerParams(use_tc_tiling_on_sc=False),
    scratch_types=[
        pltpu.VMEM((MAXCH * CHUNK,), jnp.int32),
        pltpu.VMEM((MAXCH, CHUNK), jnp.int32),
        pltpu.VMEM((12, CHUNK, F_OUT), jnp.float32),
        pltpu.VMEM((112, F_OUT), jnp.float32),
        pltpu.VMEM_SHARED((N_PAD, F_OUT), jnp.float32),
        pltpu.SemaphoreType.DMA,
        pltpu.SemaphoreType.DMA,
    ],
)
def _sc_spmm(tab_hbm, ei_hbm, rowp_hbm, acc0_out, acc1_out,
             col_v, rowp_v, buf, stage_v, acc, semg, sems):
    c = lax.axis_index("c")
    s = lax.axis_index("s")
    wid, cbase, nch = _tile_work()
    ebase = cbase * CHUNK
    pltpu.sync_copy(ei_hbm.at[1, pl.ds(ebase, NCH_BASE * CHUNK)],
                    col_v.at[pl.ds(0, NCH_BASE * CHUNK)])
    pltpu.sync_copy(rowp_hbm.at[pl.ds(cbase, NCH_BASE), :],
                    rowp_v.at[pl.ds(0, NCH_BASE), :])

    @pl.when(wid < NCH_XTRA)
    def _():
        pltpu.sync_copy(ei_hbm.at[1, pl.ds(ebase + NCH_BASE * CHUNK, CHUNK)],
                        col_v.at[pl.ds(NCH_BASE * CHUNK, CHUNK)])
        pltpu.sync_copy(rowp_hbm.at[pl.ds(cbase + NCH_BASE, 1), :],
                        rowp_v.at[pl.ds(NCH_BASE, 1), :])

    def zfill(i, carry):
        stage_v[i, pl.ds(0, 16)] = jnp.zeros((16,), jnp.float32)
        return carry

    lax.fori_loop(0, 112, zfill, 0)

    def zcopy(i, carry):
        pltpu.sync_copy(stage_v, acc.at[pl.ds(s * STRIPE + i * 112, 112), :])
        return carry

    lax.fori_loop(0, STRIPE // 112, zcopy, 0)
    plsc.subcore_barrier()

    def _gather(j, slot):
        pltpu.async_copy(tab_hbm.at[col_v.at[pl.ds(j * CHUNK, CHUNK)]],
                         buf.at[slot], semg)

    def _wait_gather(j, slot):
        pltpu.make_async_copy(tab_hbm.at[col_v.at[pl.ds(j * CHUNK, CHUNK)]],
                              buf.at[slot], semg).wait()

    def _wait_scatter(j, slot):
        pltpu.make_async_copy(buf.at[slot], acc.at[rowp_v.at[j]], sems).wait()

    for k in range(9):
        _gather(k, k)

    def body(j, carry):
        @pl.when(j >= 3)
        def _():
            _wait_scatter(j - 3, (j - 3) % 12)

        @pl.when(j + 9 < nch)
        def _():
            _gather(j + 9, (j + 9) % 12)

        _wait_gather(j, j % 12)
        pltpu.async_copy(buf.at[j % 12], acc.at[rowp_v.at[j]], sems, add=True)
        return carry

    lax.fori_loop(0, nch, body, 0)
    for k in range(3):
        _wait_scatter(nch - 3 + k, (nch - 3 + k) % 12)
    plsc.subcore_barrier()

    @pl.when(c == 0)
    def _():
        def dump0(i, carry):
            pltpu.sync_copy(acc.at[pl.ds(s * STRIPE + i * 112, 112), :],
                            stage_v)
            pltpu.sync_copy(stage_v,
                            acc0_out.at[pl.ds(s * STRIPE + i * 112, 112), :])
            return carry

        lax.fori_loop(0, STRIPE // 112, dump0, 0)

    @pl.when(c == 1)
    def _():
        def dump1(i, carry):
            pltpu.sync_copy(acc.at[pl.ds(s * STRIPE + i * 112, 112), :],
                            stage_v)
            pltpu.sync_copy(stage_v,
                            acc1_out.at[pl.ds(s * STRIPE + i * 112, 112), :])
            return carry

        lax.fori_loop(0, STRIPE // 112, dump1, 0)


_BN = 5000


NR = N_PAD // 8
NRX = N_NODES // 8
_RB = NR // 8
_GRID = 8


def _rep_mat():
    return jnp.repeat(jnp.eye(8, dtype=jnp.float32), F_OUT, axis=1)


def _dis128(d0_ref, d1_ref):
    deg = d0_ref[...] + d1_ref[...]
    dis8 = jnp.where(deg > 0, lax.rsqrt(deg), 0.0)
    return jnp.dot(dis8, _rep_mat(), preferred_element_type=jnp.float32)


def _tc_m_body(x_ref, w_ref, y0_ref, y1_ref, y2_ref):
    xb = x_ref[...]
    w = w_ref[...]
    y0_ref[...] = jnp.dot(xb, w[0], preferred_element_type=jnp.float32)
    y1_ref[...] = jnp.dot(xb, w[1], preferred_element_type=jnp.float32)
    y2_ref[...] = jnp.dot(xb, w[2], preferred_element_type=jnp.float32)


def _tc_s_body(d0_ref, d1_ref, y2_ref, g1_ref):
    g1_ref[...] = _dis128(d0_ref, d1_ref) * y2_ref[...]


def _tc_b_body(y1_ref, u0_ref, u1_ref, d0_ref, d1_ref, z_ref):
    dis = _dis128(d0_ref, d1_ref)
    u = u0_ref[...] + u1_ref[...]
    z_ref[...] = dis * y1_ref[...] - 2.0 * (dis * dis) * u


def _tc_c_body(y0_ref, v0_ref, v1_ref, d0_ref, d1_ref, b_ref, o_ref):
    dis = _dis128(d0_ref, d1_ref)
    v = v0_ref[...] + v1_ref[...]
    o_ref[...] = y0_ref[...] - dis * v + b_ref[...]


def _p_spec(width):
    return pl.BlockSpec((_RB, width), lambda i: (i, 0))


_PK = jax.ShapeDtypeStruct((NR, 128), jnp.float32)

_tc_m = pl.pallas_call(
    _tc_m_body,
    grid=(_GRID,),
    in_specs=[
        _p_spec(8 * F_IN),
        pl.BlockSpec((3, 8 * F_IN, 128), lambda i: (0, 0, 0)),
    ],
    out_specs=[_p_spec(128), _p_spec(128), _p_spec(128)],
    out_shape=[_PK, _PK, _PK],
)

_tc_s = pl.pallas_call(
    _tc_s_body,
    grid=(_GRID,),
    in_specs=[_p_spec(8), _p_spec(8), _p_spec(128)],
    out_specs=_p_spec(128),
    out_shape=_PK,
)

_tc_b = pl.pallas_call(
    _tc_b_body,
    grid=(_GRID,),
    in_specs=[_p_spec(128), _p_spec(128), _p_spec(128), _p_spec(8),
              _p_spec(8)],
    out_specs=_p_spec(128),
    out_shape=_PK,
)

_tc_c = pl.pallas_call(
    _tc_c_body,
    grid=(_GRID,),
    in_specs=[_p_spec(128), _p_spec(128), _p_spec(128), _p_spec(8),
              _p_spec(8), pl.BlockSpec((1, 128), lambda i: (0, 0))],
    out_specs=_p_spec(128),
    out_shape=_PK,
)


@jax.jit
def kernel(x, edge_index, weight, bias):
    n = x.shape[0]
    x_p = x.reshape(NRX, 8 * F_IN)
    w = weight.reshape(weight.shape[0], F_IN, F_OUT)
    eye8 = jnp.eye(8, dtype=jnp.float32)
    wbd = jnp.stack([jnp.kron(eye8, w[0] - w[2]),
                     jnp.kron(eye8, w[1]),
                     jnp.kron(eye8, w[2])])

    deg0, deg1, rowp = _sc_degree(edge_index)
    d0_8 = deg0.reshape(NR, 8)
    d1_8 = deg1.reshape(NR, 8)

    y0, y1, y2 = _tc_m(x_p, wbd)
    g1 = _tc_s(d0_8, d1_8, y2)

    u0, u1 = _sc_spmm(g1.reshape(N_PAD, F_OUT), edge_index, rowp)
    z = _tc_b(y1, u0.reshape(NR, 128), u1.reshape(NR, 128), d0_8, d1_8)

    v0, v1 = _sc_spmm(z.reshape(N_PAD, F_OUT), edge_index, rowp)
    out_p = _tc_c(y0, v0.reshape(NR, 128), v1.reshape(NR, 128), d0_8, d1_8,
                  jnp.tile(bias, 8).reshape(1, 128))

    return out_p.reshape(N_PAD, F_OUT)[:n].reshape(n, 1, F_OUT, 1)
